# Initial kernel scaffold; baseline (speedup 1.0000x reference)
import jax
import jax.numpy as jnp
from jax.experimental import pallas as pl


def kernel(x1, x2, qkv_w, qkv_b, wo_w, wo_b, lepe_w, lepe_b):
    raise NotImplementedError("write your pallas kernel here")


# speedup vs baseline: 1.5007x; 1.5007x over previous
import functools

import numpy as np
import jax
import jax.numpy as jnp
from jax import lax
from jax.experimental import pallas as pl
from jax.experimental.pallas import tpu as pltpu

DIM = 192
QK = 192
NWIN = 8
P2 = NWIN * NWIN
WS = 28
W2 = WS * WS
NH = 8
HD = DIM // NH
TOPK = 4
KVW = 4
NKV = KVW * KVW
SCALE = QK ** (-0.5)
H = 224
BH = 14
BANDS = H // BH

_INTERPRET = False


def _pool_matrix():
    p = np.zeros((NKV, W2), np.float32)
    for a in range(KVW):
        for b in range(KVW):
            for r in range(7 * a, 7 * a + 7):
                for c in range(7 * b, 7 * b + 7):
                    p[a * KVW + b, r * WS + c] = 1.0 / 49.0
    return jnp.asarray(p)


def _stage_a_body(x1_ref, x2_ref, w_ref, b_ref, p_ref,
                  q1_ref, q2_ref, kvp1_ref, kvp2_ref,
                  qm1_ref, km1_ref, qm2_ref, km2_ref):
    w = w_ref[...]
    b = b_ref[...]
    pmat = p_ref[...]
    for x_ref, q_ref, kvp_ref, qm_ref, km_ref in (
            (x1_ref, q1_ref, kvp1_ref, qm1_ref, km1_ref),
            (x2_ref, q2_ref, kvp2_ref, qm2_ref, km2_ref)):
        t = jnp.dot(x_ref[0], w, preferred_element_type=jnp.float32) + b
        q_ref[0] = t[:, :QK]
        kvp_ref[0] = jnp.dot(pmat, t[:, QK:], preferred_element_type=jnp.float32)
        m = jnp.mean(t, axis=0, keepdims=True)
        qm_ref[0] = m[:, :QK]
        km_ref[0] = m[:, QK:2 * QK]


def _stage_a(x1w, x2w, qkv_w, qkv_b, pmat):
    f32 = jnp.float32
    outs = (
        jax.ShapeDtypeStruct((P2, W2, QK), f32),
        jax.ShapeDtypeStruct((P2, W2, QK), f32),
        jax.ShapeDtypeStruct((P2, NKV, 2 * QK), f32),
        jax.ShapeDtypeStruct((P2, NKV, 2 * QK), f32),
        jax.ShapeDtypeStruct((P2, 1, QK), f32),
        jax.ShapeDtypeStruct((P2, 1, QK), f32),
        jax.ShapeDtypeStruct((P2, 1, QK), f32),
        jax.ShapeDtypeStruct((P2, 1, QK), f32),
    )
    win = lambda i: (i, 0, 0)
    full2 = lambda i: (0, 0)
    in_specs = [
        pl.BlockSpec((1, W2, DIM), win),
        pl.BlockSpec((1, W2, DIM), win),
        pl.BlockSpec((DIM, 2 * QK + DIM), full2),
        pl.BlockSpec((1, 2 * QK + DIM), full2),
        pl.BlockSpec((NKV, W2), full2),
    ]
    out_specs = (
        pl.BlockSpec((1, W2, QK), win),
        pl.BlockSpec((1, W2, QK), win),
        pl.BlockSpec((1, NKV, 2 * QK), win),
        pl.BlockSpec((1, NKV, 2 * QK), win),
        pl.BlockSpec((1, 1, QK), win),
        pl.BlockSpec((1, 1, QK), win),
        pl.BlockSpec((1, 1, QK), win),
        pl.BlockSpec((1, 1, QK), win),
    )
    return pl.pallas_call(
        _stage_a_body,
        grid=(P2,),
        in_specs=in_specs,
        out_specs=out_specs,
        out_shape=outs,
        interpret=_INTERPRET,
    )(x1w, x2w, qkv_w, qkv_b.reshape(1, -1), pmat)


def _topk_rows(logits):
    colid = lax.broadcasted_iota(jnp.int32, (P2, P2), 1)
    idxs = []
    for _ in range(TOPK):
        mx = jnp.max(logits, axis=1, keepdims=True)
        cand = jnp.where(logits >= mx, colid, P2)
        am = jnp.min(cand, axis=1, keepdims=True)
        idxs.append(am)
        logits = jnp.where(colid == am, -jnp.float32(np.inf), logits)
    return jnp.concatenate(idxs, axis=1)


def _router_body(qm1_ref, km1_ref, qm2_ref, km2_ref, idx1_ref, idx2_ref):
    rowid = lax.broadcasted_iota(jnp.int32, (P2, P2), 0)
    colid = lax.broadcasted_iota(jnp.int32, (P2, P2), 1)
    diag = rowid == colid
    dn = (((1,), (1,)), ((), ()))
    l1 = lax.dot_general(qm2_ref[...] * SCALE, km1_ref[...], dn,
                         preferred_element_type=jnp.float32)
    l1 = jnp.where(diag, 1.0, l1)
    idx1_ref[...] = _topk_rows(l1)
    l2 = lax.dot_general(qm1_ref[...] * SCALE, km2_ref[...], dn,
                         preferred_element_type=jnp.float32)
    l2 = jnp.where(diag, 1.0, l2)
    idx2_ref[...] = _topk_rows(l2)


def _router(qm1, km1, qm2, km2):
    outs = (jax.ShapeDtypeStruct((P2, TOPK), jnp.int32),
            jax.ShapeDtypeStruct((P2, TOPK), jnp.int32))
    spec = pl.BlockSpec((P2, QK), lambda: (0, 0))
    ospec = pl.BlockSpec((P2, TOPK), lambda: (0, 0))
    return pl.pallas_call(
        _router_body,
        grid=(),
        in_specs=[spec] * 4,
        out_specs=(ospec, ospec),
        out_shape=outs,
        interpret=_INTERPRET,
    )(qm1, km1, qm2, km2)


def _attend_one(q_ref, kvp_ref, idx_ref, a_ref, w):
    rows = []
    for j in range(TOPK):
        r = idx_ref[w, j]
        rows.append(kvp_ref[r])
    kv = jnp.concatenate(rows, axis=0)
    k_sel = kv[:, :QK]
    v_sel = kv[:, QK:]
    q = q_ref[0] * SCALE
    dn = (((1,), (1,)), ((), ()))
    outs = []
    for h in range(NH):
        qh = q[:, h * HD:(h + 1) * HD]
        kh = k_sel[:, h * HD:(h + 1) * HD]
        vh = v_sel[:, h * HD:(h + 1) * HD]
        lg = lax.dot_general(qh, kh, dn, preferred_element_type=jnp.float32)
        m = jnp.max(lg, axis=1, keepdims=True)
        e = jnp.exp(lg - m)
        p = e / jnp.sum(e, axis=1, keepdims=True)
        outs.append(jnp.dot(p, vh, preferred_element_type=jnp.float32))
    a_ref[0] = jnp.concatenate(outs, axis=1)


def _attn_body(idx1_ref, idx2_ref, q1_ref, q2_ref, kvp1_ref, kvp2_ref,
               a1_ref, a2_ref):
    w = pl.program_id(0)
    _attend_one(q2_ref, kvp1_ref, idx1_ref, a1_ref, w)
    _attend_one(q1_ref, kvp2_ref, idx2_ref, a2_ref, w)


def _attention(q1, q2, kvp1, kvp2, idx1, idx2):
    f32 = jnp.float32
    win = lambda i: (i, 0, 0)
    full3 = lambda i: (0, 0, 0)
    outs = (jax.ShapeDtypeStruct((P2, W2, DIM), f32),
            jax.ShapeDtypeStruct((P2, W2, DIM), f32))
    in_specs = [
        pl.BlockSpec(memory_space=pltpu.SMEM),
        pl.BlockSpec(memory_space=pltpu.SMEM),
        pl.BlockSpec((1, W2, QK), win),
        pl.BlockSpec((1, W2, QK), win),
        pl.BlockSpec((P2, NKV, 2 * QK), full3),
        pl.BlockSpec((P2, NKV, 2 * QK), full3),
    ]
    out_specs = (pl.BlockSpec((1, W2, DIM), win),
                 pl.BlockSpec((1, W2, DIM), win))
    return pl.pallas_call(
        _attn_body,
        grid=(P2,),
        in_specs=in_specs,
        out_specs=out_specs,
        out_shape=outs,
        interpret=_INTERPRET,
    )(idx1, idx2, q1, q2, kvp1, kvp2)


def _lepe_band(i, x_ref, xu_ref, xd_ref, wv, w9, lepe_b):
    vb = jnp.dot(x_ref[0].reshape(BH * H, DIM), wv,
                 preferred_element_type=jnp.float32).reshape(BH, H, DIM)
    vu = jnp.dot(xu_ref[0, 0], wv, preferred_element_type=jnp.float32)
    vu = jnp.where(i == 0, 0.0, vu)[None]
    vd = jnp.dot(xd_ref[0, 0], wv, preferred_element_type=jnp.float32)
    vd = jnp.where(i == BANDS - 1, 0.0, vd)[None]
    v_ext = jnp.concatenate([vu, vb, vd], axis=0)
    zc = jnp.zeros((BH + 2, 1, DIM), jnp.float32)
    v_pad = jnp.concatenate([zc, v_ext, zc], axis=1)
    acc = jnp.broadcast_to(lepe_b.reshape(1, 1, DIM), (BH, H, DIM))
    for ky in range(3):
        for kx in range(3):
            acc = acc + w9[ky, kx] * v_pad[ky:ky + BH, kx:kx + H, :]
    return acc


def _final_body(a_ref, x_ref, xu_ref, xd_ref,
                wv_ref, w9_ref, lb_ref, wo_ref, wob_ref, o_ref):
    i = pl.program_id(0)
    lepe = _lepe_band(i, x_ref, xu_ref, xd_ref, wv_ref[...], w9_ref[...],
                      lb_ref[...])
    s = a_ref[0].reshape(BH * H, DIM) + lepe.reshape(BH * H, DIM)
    o = jnp.dot(s, wo_ref[...], preferred_element_type=jnp.float32) + wob_ref[...]
    o_ref[0] = o.reshape(BH, H, DIM)


def _final(a_img, x, wv, w9, lepe_b, wo_w, wo_b):
    f32 = jnp.float32
    up = lambda i: (0, jnp.maximum(i * BH - 1, 0), 0, 0)
    dn = lambda i: (0, jnp.minimum((i + 1) * BH, H - 1), 0, 0)
    outs = jax.ShapeDtypeStruct((BANDS, BH, H, DIM), f32)
    bspec = pl.BlockSpec((1, BH, H, DIM), lambda i: (i, 0, 0, 0))
    xspec = pl.BlockSpec((1, BH, H, DIM), lambda i: (0, i, 0, 0))
    rspec_u = pl.BlockSpec((1, 1, H, DIM), up)
    rspec_d = pl.BlockSpec((1, 1, H, DIM), dn)
    full2 = lambda i: (0, 0)
    in_specs = [
        bspec,
        xspec, rspec_u, rspec_d,
        pl.BlockSpec((DIM, DIM), full2),
        pl.BlockSpec((3, 3, DIM), lambda i: (0, 0, 0)),
        pl.BlockSpec((1, DIM), full2),
        pl.BlockSpec((DIM, DIM), full2),
        pl.BlockSpec((1, DIM), full2),
    ]
    return pl.pallas_call(
        _final_body,
        grid=(BANDS,),
        in_specs=in_specs,
        out_specs=bspec,
        out_shape=outs,
        interpret=_INTERPRET,
    )(a_img.reshape(BANDS, BH, H, DIM), x, x, x, wv, w9,
      lepe_b.reshape(1, -1), wo_w, wo_b.reshape(1, -1))


def _window(x):
    c = x.shape[-1]
    x = x.reshape(NWIN, WS, NWIN, WS, c).transpose(0, 2, 1, 3, 4)
    return x.reshape(P2, W2, c)


def _unwindow_bands(a):
    c = a.shape[-1]
    a = a.reshape(NWIN, NWIN, WS, WS, c).transpose(0, 2, 1, 3, 4)
    return a.reshape(NWIN, WS, NWIN * WS, c)


def kernel(x1, x2, qkv_w, qkv_b, wo_w, wo_b, lepe_w, lepe_b):
    pmat = _pool_matrix()
    x1w = _window(x1)
    x2w = _window(x2)
    q1, q2, kvp1, kvp2, qm1, km1, qm2, km2 = _stage_a(
        x1w, x2w, qkv_w, qkv_b, pmat)
    idx1, idx2 = _router(qm1.reshape(P2, QK), km1.reshape(P2, QK),
                         qm2.reshape(P2, QK), km2.reshape(P2, QK))
    a1, a2 = _attention(q1, q2, kvp1, kvp2, idx1, idx2)
    a1b = _unwindow_bands(a1)
    a2b = _unwindow_bands(a2)
    wv = qkv_w[:, 2 * QK:]
    w9 = lepe_w[:, 0].transpose(1, 2, 0)
    x1b = x1.reshape(1, H, H, DIM)
    x2b = x2.reshape(1, H, H, DIM)
    o1 = _final(a1b, x1b, wv, w9, lepe_b, wo_w, wo_b)
    o2 = _final(a2b, x2b, wv, w9, lepe_b, wo_w, wo_b)
    return o1.reshape(1, H, H, DIM), o2.reshape(1, H, H, DIM)


# speedup vs baseline: 1.8044x; 1.2024x over previous
import functools

import numpy as np
import jax
import jax.numpy as jnp
from jax import lax
from jax.experimental import pallas as pl
from jax.experimental.pallas import tpu as pltpu

DIM = 192
QK = 192
NWIN = 8
P2 = NWIN * NWIN
WS = 28
W2 = WS * WS
NH = 8
HD = DIM // NH
TOPK = 4
KVW = 4
NKV = KVW * KVW
SCALE = QK ** (-0.5)
H = 224
BH = 14
BANDS = H // BH

_INTERPRET = False


def _pool_matrix():
    p = np.zeros((NKV, W2), np.float32)
    for a in range(KVW):
        for b in range(KVW):
            for r in range(7 * a, 7 * a + 7):
                for c in range(7 * b, 7 * b + 7):
                    p[a * KVW + b, r * WS + c] = 1.0 / 49.0
    return jnp.asarray(p)


def _stage_a_body(x1_ref, x2_ref, w_ref, b_ref, p_ref,
                  q1_ref, q2_ref, kvp1_ref, kvp2_ref,
                  qm1_ref, km1_ref, qm2_ref, km2_ref):
    w = w_ref[...]
    b = b_ref[...]
    pmat = p_ref[...]
    for x_ref, q_ref, kvp_ref, qm_ref, km_ref in (
            (x1_ref, q1_ref, kvp1_ref, qm1_ref, km1_ref),
            (x2_ref, q2_ref, kvp2_ref, qm2_ref, km2_ref)):
        xw = x_ref[0, :, 0].reshape(W2, DIM)
        t = jnp.dot(xw, w, preferred_element_type=jnp.float32) + b
        q_ref[0] = t[:, :QK]
        kvp_ref[0] = jnp.dot(pmat, t[:, QK:], preferred_element_type=jnp.float32)
        m = jnp.mean(t, axis=0, keepdims=True)
        qm_ref[0] = m[:, :QK]
        km_ref[0] = m[:, QK:2 * QK]


def _stage_a(x1w, x2w, qkv_w, qkv_b, pmat):
    f32 = jnp.float32
    outs = (
        jax.ShapeDtypeStruct((P2, W2, QK), f32),
        jax.ShapeDtypeStruct((P2, W2, QK), f32),
        jax.ShapeDtypeStruct((P2, NKV, 2 * QK), f32),
        jax.ShapeDtypeStruct((P2, NKV, 2 * QK), f32),
        jax.ShapeDtypeStruct((P2, 1, QK), f32),
        jax.ShapeDtypeStruct((P2, 1, QK), f32),
        jax.ShapeDtypeStruct((P2, 1, QK), f32),
        jax.ShapeDtypeStruct((P2, 1, QK), f32),
    )
    win = lambda i: (i, 0, 0)
    xwin = lambda i: (i // NWIN, 0, i % NWIN, 0, 0)
    full2 = lambda i: (0, 0)
    in_specs = [
        pl.BlockSpec((1, WS, 1, WS, DIM), xwin),
        pl.BlockSpec((1, WS, 1, WS, DIM), xwin),
        pl.BlockSpec((DIM, 2 * QK + DIM), full2),
        pl.BlockSpec((1, 2 * QK + DIM), full2),
        pl.BlockSpec((NKV, W2), full2),
    ]
    out_specs = (
        pl.BlockSpec((1, W2, QK), win),
        pl.BlockSpec((1, W2, QK), win),
        pl.BlockSpec((1, NKV, 2 * QK), win),
        pl.BlockSpec((1, NKV, 2 * QK), win),
        pl.BlockSpec((1, 1, QK), win),
        pl.BlockSpec((1, 1, QK), win),
        pl.BlockSpec((1, 1, QK), win),
        pl.BlockSpec((1, 1, QK), win),
    )
    return pl.pallas_call(
        _stage_a_body,
        grid=(P2,),
        in_specs=in_specs,
        out_specs=out_specs,
        out_shape=outs,
        interpret=_INTERPRET,
    )(x1w, x2w, qkv_w, qkv_b.reshape(1, -1), pmat)


def _topk_rows(logits):
    colid = lax.broadcasted_iota(jnp.int32, (P2, P2), 1)
    idxs = []
    for _ in range(TOPK):
        mx = jnp.max(logits, axis=1, keepdims=True)
        cand = jnp.where(logits >= mx, colid, P2)
        am = jnp.min(cand, axis=1, keepdims=True)
        idxs.append(am)
        logits = jnp.where(colid == am, -jnp.float32(np.inf), logits)
    return jnp.concatenate(idxs, axis=1)


def _router_body(qm1_ref, km1_ref, qm2_ref, km2_ref, idx1_ref, idx2_ref):
    rowid = lax.broadcasted_iota(jnp.int32, (P2, P2), 0)
    colid = lax.broadcasted_iota(jnp.int32, (P2, P2), 1)
    diag = rowid == colid
    dn = (((1,), (1,)), ((), ()))
    l1 = lax.dot_general(qm2_ref[...] * SCALE, km1_ref[...], dn,
                         preferred_element_type=jnp.float32)
    l1 = jnp.where(diag, 1.0, l1)
    idx1_ref[...] = _topk_rows(l1)
    l2 = lax.dot_general(qm1_ref[...] * SCALE, km2_ref[...], dn,
                         preferred_element_type=jnp.float32)
    l2 = jnp.where(diag, 1.0, l2)
    idx2_ref[...] = _topk_rows(l2)


def _router(qm1, km1, qm2, km2):
    outs = (jax.ShapeDtypeStruct((P2, TOPK), jnp.int32),
            jax.ShapeDtypeStruct((P2, TOPK), jnp.int32))
    spec = pl.BlockSpec((P2, QK), lambda: (0, 0))
    ospec = pl.BlockSpec((P2, TOPK), lambda: (0, 0))
    return pl.pallas_call(
        _router_body,
        grid=(),
        in_specs=[spec] * 4,
        out_specs=(ospec, ospec),
        out_shape=outs,
        interpret=_INTERPRET,
    )(qm1, km1, qm2, km2)


def _attend_one(q_ref, kvp_ref, idx_ref, a_ref, w):
    rows = []
    for j in range(TOPK):
        r = idx_ref[w, j]
        rows.append(kvp_ref[r])
    kv = jnp.concatenate(rows, axis=0)
    k_sel = kv[:, :QK]
    v_sel = kv[:, QK:]
    q = q_ref[0] * SCALE
    dn = (((1,), (1,)), ((), ()))
    outs = []
    for h in range(NH):
        qh = q[:, h * HD:(h + 1) * HD]
        kh = k_sel[:, h * HD:(h + 1) * HD]
        vh = v_sel[:, h * HD:(h + 1) * HD]
        lg = lax.dot_general(qh, kh, dn, preferred_element_type=jnp.float32)
        m = jnp.max(lg, axis=1, keepdims=True)
        e = jnp.exp(lg - m)
        p = e / jnp.sum(e, axis=1, keepdims=True)
        outs.append(jnp.dot(p, vh, preferred_element_type=jnp.float32))
    a_ref[0, :, 0] = jnp.concatenate(outs, axis=1).reshape(WS, WS, DIM)


def _attn_body(idx1_ref, idx2_ref, q1_ref, q2_ref, kvp1_ref, kvp2_ref,
               a1_ref, a2_ref):
    w = pl.program_id(0)
    _attend_one(q2_ref, kvp1_ref, idx1_ref, a1_ref, w)
    _attend_one(q1_ref, kvp2_ref, idx2_ref, a2_ref, w)


def _attention(q1, q2, kvp1, kvp2, idx1, idx2):
    f32 = jnp.float32
    win = lambda i: (i, 0, 0)
    awin = lambda i: (i // NWIN, 0, i % NWIN, 0, 0)
    full3 = lambda i: (0, 0, 0)
    outs = (jax.ShapeDtypeStruct((NWIN, WS, NWIN, WS, DIM), f32),
            jax.ShapeDtypeStruct((NWIN, WS, NWIN, WS, DIM), f32))
    in_specs = [
        pl.BlockSpec(memory_space=pltpu.SMEM),
        pl.BlockSpec(memory_space=pltpu.SMEM),
        pl.BlockSpec((1, W2, QK), win),
        pl.BlockSpec((1, W2, QK), win),
        pl.BlockSpec((P2, NKV, 2 * QK), full3),
        pl.BlockSpec((P2, NKV, 2 * QK), full3),
    ]
    out_specs = (pl.BlockSpec((1, WS, 1, WS, DIM), awin),
                 pl.BlockSpec((1, WS, 1, WS, DIM), awin))
    return pl.pallas_call(
        _attn_body,
        grid=(P2,),
        in_specs=in_specs,
        out_specs=out_specs,
        out_shape=outs,
        interpret=_INTERPRET,
    )(idx1, idx2, q1, q2, kvp1, kvp2)


def _lepe_band(i, x_ref, xu_ref, xd_ref, wv, w9, lepe_b):
    vb = jnp.dot(x_ref[0].reshape(BH * H, DIM), wv,
                 preferred_element_type=jnp.float32).reshape(BH, H, DIM)
    vu = jnp.dot(xu_ref[0, 0], wv, preferred_element_type=jnp.float32)
    vu = jnp.where(i == 0, 0.0, vu)[None]
    vd = jnp.dot(xd_ref[0, 0], wv, preferred_element_type=jnp.float32)
    vd = jnp.where(i == BANDS - 1, 0.0, vd)[None]
    v_ext = jnp.concatenate([vu, vb, vd], axis=0)
    zc = jnp.zeros((BH + 2, 1, DIM), jnp.float32)
    v_pad = jnp.concatenate([zc, v_ext, zc], axis=1)
    acc = jnp.broadcast_to(lepe_b.reshape(1, 1, DIM), (BH, H, DIM))
    for ky in range(3):
        for kx in range(3):
            acc = acc + w9[ky, kx] * v_pad[ky:ky + BH, kx:kx + H, :]
    return acc


def _final_body(a_ref, x_ref, xu_ref, xd_ref,
                wv_ref, w9_ref, lb_ref, wo_ref, wob_ref, o_ref):
    i = pl.program_id(0)
    lepe = _lepe_band(i, x_ref, xu_ref, xd_ref, wv_ref[...], w9_ref[...],
                      lb_ref[...])
    s = a_ref[0].reshape(BH * H, DIM) + lepe.reshape(BH * H, DIM)
    o = jnp.dot(s, wo_ref[...], preferred_element_type=jnp.float32) + wob_ref[...]
    o_ref[0] = o.reshape(BH, H, DIM)


def _final(a_img, x, wv, w9, lepe_b, wo_w, wo_b):
    f32 = jnp.float32
    up = lambda i: (0, jnp.maximum(i * BH - 1, 0), 0, 0)
    dn = lambda i: (0, jnp.minimum((i + 1) * BH, H - 1), 0, 0)
    outs = jax.ShapeDtypeStruct((BANDS, BH, H, DIM), f32)
    bspec = pl.BlockSpec((1, BH, H, DIM), lambda i: (i, 0, 0, 0))
    xspec = pl.BlockSpec((1, BH, H, DIM), lambda i: (0, i, 0, 0))
    rspec_u = pl.BlockSpec((1, 1, H, DIM), up)
    rspec_d = pl.BlockSpec((1, 1, H, DIM), dn)
    full2 = lambda i: (0, 0)
    in_specs = [
        bspec,
        xspec, rspec_u, rspec_d,
        pl.BlockSpec((DIM, DIM), full2),
        pl.BlockSpec((3, 3, DIM), lambda i: (0, 0, 0)),
        pl.BlockSpec((1, DIM), full2),
        pl.BlockSpec((DIM, DIM), full2),
        pl.BlockSpec((1, DIM), full2),
    ]
    return pl.pallas_call(
        _final_body,
        grid=(BANDS,),
        in_specs=in_specs,
        out_specs=bspec,
        out_shape=outs,
        interpret=_INTERPRET,
    )(a_img.reshape(BANDS, BH, H, DIM), x, x, x, wv, w9,
      lepe_b.reshape(1, -1), wo_w, wo_b.reshape(1, -1))


def kernel(x1, x2, qkv_w, qkv_b, wo_w, wo_b, lepe_w, lepe_b):
    pmat = _pool_matrix()
    x1w = x1.reshape(NWIN, WS, NWIN, WS, DIM)
    x2w = x2.reshape(NWIN, WS, NWIN, WS, DIM)
    q1, q2, kvp1, kvp2, qm1, km1, qm2, km2 = _stage_a(
        x1w, x2w, qkv_w, qkv_b, pmat)
    idx1, idx2 = _router(qm1.reshape(P2, QK), km1.reshape(P2, QK),
                         qm2.reshape(P2, QK), km2.reshape(P2, QK))
    a1, a2 = _attention(q1, q2, kvp1, kvp2, idx1, idx2)
    a1b = a1.reshape(BANDS, BH, H, DIM)
    a2b = a2.reshape(BANDS, BH, H, DIM)
    wv = qkv_w[:, 2 * QK:]
    w9 = lepe_w[:, 0].transpose(1, 2, 0)
    x1b = x1.reshape(1, H, H, DIM)
    x2b = x2.reshape(1, H, H, DIM)
    o1 = _final(a1b, x1b, wv, w9, lepe_b, wo_w, wo_b)
    o2 = _final(a2b, x2b, wv, w9, lepe_b, wo_w, wo_b)
    return o1.reshape(1, H, H, DIM), o2.reshape(1, H, H, DIM)


# speedup vs baseline: 1.8852x; 1.0448x over previous
import functools

import numpy as np
import jax
import jax.numpy as jnp
from jax import lax
from jax.experimental import pallas as pl
from jax.experimental.pallas import tpu as pltpu

DIM = 192
QK = 192
NWIN = 8
P2 = NWIN * NWIN
WS = 28
W2 = WS * WS
NH = 8
HD = DIM // NH
TOPK = 4
KVW = 4
NKV = KVW * KVW
SCALE = QK ** (-0.5)
H = 224
BH = 14
BANDS = H // BH

_INTERPRET = False


def _pool_matrix():
    p = np.zeros((NKV, W2), np.float32)
    for a in range(KVW):
        for b in range(KVW):
            for r in range(7 * a, 7 * a + 7):
                for c in range(7 * b, 7 * b + 7):
                    p[a * KVW + b, r * WS + c] = 1.0 / 49.0
    return jnp.asarray(p)


def _stage_a_body(x1_ref, x2_ref, w_ref, b_ref, p_ref,
                  q1_ref, q2_ref, kvp1_ref, kvp2_ref,
                  qm1_ref, km1_ref, qm2_ref, km2_ref):
    w = w_ref[...]
    b = b_ref[...]
    pmat = p_ref[...]
    for x_ref, q_ref, kvp_ref, qm_ref, km_ref in (
            (x1_ref, q1_ref, kvp1_ref, qm1_ref, km1_ref),
            (x2_ref, q2_ref, kvp2_ref, qm2_ref, km2_ref)):
        band = x_ref[0]
        qs, kvps, qms, kms = [], [], [], []
        for c in range(NWIN):
            xw = band[:, c * WS:(c + 1) * WS, :].reshape(W2, DIM)
            t = jnp.dot(xw, w, preferred_element_type=jnp.float32) + b
            qs.append(t[:, :QK])
            kvps.append(jnp.dot(pmat, t[:, QK:],
                                preferred_element_type=jnp.float32))
            m = jnp.mean(t, axis=0, keepdims=True)
            qms.append(m[:, :QK])
            kms.append(m[:, QK:2 * QK])
        q_ref[...] = jnp.stack(qs)
        kvp_ref[...] = jnp.stack(kvps)
        qm_ref[...] = jnp.stack(qms)
        km_ref[...] = jnp.stack(kms)


def _stage_a(x1, x2, qkv_w, qkv_b, pmat):
    f32 = jnp.float32
    outs = (
        jax.ShapeDtypeStruct((P2, W2, QK), f32),
        jax.ShapeDtypeStruct((P2, W2, QK), f32),
        jax.ShapeDtypeStruct((P2, NKV, 2 * QK), f32),
        jax.ShapeDtypeStruct((P2, NKV, 2 * QK), f32),
        jax.ShapeDtypeStruct((P2, 1, QK), f32),
        jax.ShapeDtypeStruct((P2, 1, QK), f32),
        jax.ShapeDtypeStruct((P2, 1, QK), f32),
        jax.ShapeDtypeStruct((P2, 1, QK), f32),
    )
    band = lambda i: (i, 0, 0)
    xband = lambda i: (0, i, 0, 0)
    full2 = lambda i: (0, 0)
    in_specs = [
        pl.BlockSpec((1, WS, H, DIM), xband),
        pl.BlockSpec((1, WS, H, DIM), xband),
        pl.BlockSpec((DIM, 2 * QK + DIM), full2),
        pl.BlockSpec((1, 2 * QK + DIM), full2),
        pl.BlockSpec((NKV, W2), full2),
    ]
    out_specs = (
        pl.BlockSpec((NWIN, W2, QK), band),
        pl.BlockSpec((NWIN, W2, QK), band),
        pl.BlockSpec((NWIN, NKV, 2 * QK), band),
        pl.BlockSpec((NWIN, NKV, 2 * QK), band),
        pl.BlockSpec((NWIN, 1, QK), band),
        pl.BlockSpec((NWIN, 1, QK), band),
        pl.BlockSpec((NWIN, 1, QK), band),
        pl.BlockSpec((NWIN, 1, QK), band),
    )
    return pl.pallas_call(
        _stage_a_body,
        grid=(NWIN,),
        in_specs=in_specs,
        out_specs=out_specs,
        out_shape=outs,
        interpret=_INTERPRET,
    )(x1, x2, qkv_w, qkv_b.reshape(1, -1), pmat)


def _topk_rows(logits):
    colid = lax.broadcasted_iota(jnp.int32, (P2, P2), 1)
    idxs = []
    for _ in range(TOPK):
        mx = jnp.max(logits, axis=1, keepdims=True)
        cand = jnp.where(logits >= mx, colid, P2)
        am = jnp.min(cand, axis=1, keepdims=True)
        idxs.append(am)
        logits = jnp.where(colid == am, -jnp.float32(np.inf), logits)
    return jnp.concatenate(idxs, axis=1)


def _router_body(qm1_ref, km1_ref, qm2_ref, km2_ref, idx1_ref, idx2_ref):
    rowid = lax.broadcasted_iota(jnp.int32, (P2, P2), 0)
    colid = lax.broadcasted_iota(jnp.int32, (P2, P2), 1)
    diag = rowid == colid
    dn = (((1,), (1,)), ((), ()))
    l1 = lax.dot_general(qm2_ref[...] * SCALE, km1_ref[...], dn,
                         preferred_element_type=jnp.float32)
    l1 = jnp.where(diag, 1.0, l1)
    idx1_ref[...] = _topk_rows(l1)
    l2 = lax.dot_general(qm1_ref[...] * SCALE, km2_ref[...], dn,
                         preferred_element_type=jnp.float32)
    l2 = jnp.where(diag, 1.0, l2)
    idx2_ref[...] = _topk_rows(l2)


def _router(qm1, km1, qm2, km2):
    outs = (jax.ShapeDtypeStruct((P2, TOPK), jnp.int32),
            jax.ShapeDtypeStruct((P2, TOPK), jnp.int32))
    spec = pl.BlockSpec((P2, QK), lambda: (0, 0))
    ospec = pl.BlockSpec((P2, TOPK), lambda: (0, 0))
    return pl.pallas_call(
        _router_body,
        grid=(),
        in_specs=[spec] * 4,
        out_specs=(ospec, ospec),
        out_shape=outs,
        interpret=_INTERPRET,
    )(qm1, km1, qm2, km2)


def _attend_one(q_ref, kvp_ref, idx_ref, wi, wslot):
    rows = []
    for j in range(TOPK):
        r = idx_ref[wi, j]
        rows.append(kvp_ref[r])
    kv = jnp.concatenate(rows, axis=0)
    k_sel = kv[:, :QK]
    v_sel = kv[:, QK:]
    q = q_ref[wslot] * SCALE
    dn = (((1,), (1,)), ((), ()))
    outs = []
    for h in range(NH):
        qh = q[:, h * HD:(h + 1) * HD]
        kh = k_sel[:, h * HD:(h + 1) * HD]
        vh = v_sel[:, h * HD:(h + 1) * HD]
        lg = lax.dot_general(qh, kh, dn, preferred_element_type=jnp.float32)
        m = jnp.max(lg, axis=1, keepdims=True)
        e = jnp.exp(lg - m)
        p = e / jnp.sum(e, axis=1, keepdims=True)
        outs.append(jnp.dot(p, vh, preferred_element_type=jnp.float32))
    return jnp.concatenate(outs, axis=1).reshape(WS, WS, DIM)


def _attn_body(idx1_ref, idx2_ref, q1_ref, q2_ref, kvp1_ref, kvp2_ref,
               a1_ref, a2_ref):
    i = pl.program_id(0)
    for q_ref, kvp_ref, idx_ref, a_ref in (
            (q2_ref, kvp1_ref, idx1_ref, a1_ref),
            (q1_ref, kvp2_ref, idx2_ref, a2_ref)):
        wins = [_attend_one(q_ref, kvp_ref, idx_ref, i * NWIN + c, c)
                for c in range(NWIN)]
        a_ref[0] = jnp.concatenate(wins, axis=1)


def _attention(q1, q2, kvp1, kvp2, idx1, idx2):
    f32 = jnp.float32
    band = lambda i: (i, 0, 0)
    aband = lambda i: (0, i, 0, 0)
    full3 = lambda i: (0, 0, 0)
    outs = (jax.ShapeDtypeStruct((1, H, H, DIM), f32),
            jax.ShapeDtypeStruct((1, H, H, DIM), f32))
    in_specs = [
        pl.BlockSpec(memory_space=pltpu.SMEM),
        pl.BlockSpec(memory_space=pltpu.SMEM),
        pl.BlockSpec((NWIN, W2, QK), band),
        pl.BlockSpec((NWIN, W2, QK), band),
        pl.BlockSpec((P2, NKV, 2 * QK), full3),
        pl.BlockSpec((P2, NKV, 2 * QK), full3),
    ]
    out_specs = (pl.BlockSpec((1, WS, H, DIM), aband),
                 pl.BlockSpec((1, WS, H, DIM), aband))
    return pl.pallas_call(
        _attn_body,
        grid=(NWIN,),
        in_specs=in_specs,
        out_specs=out_specs,
        out_shape=outs,
        interpret=_INTERPRET,
    )(idx1, idx2, q1, q2, kvp1, kvp2)


def _lepe_band(i, x_ref, xu_ref, xd_ref, wv, w9, lepe_b):
    vb = jnp.dot(x_ref[0].reshape(BH * H, DIM), wv,
                 preferred_element_type=jnp.float32).reshape(BH, H, DIM)
    vu = jnp.dot(xu_ref[0, 0], wv, preferred_element_type=jnp.float32)
    vu = jnp.where(i == 0, 0.0, vu)[None]
    vd = jnp.dot(xd_ref[0, 0], wv, preferred_element_type=jnp.float32)
    vd = jnp.where(i == BANDS - 1, 0.0, vd)[None]
    v_ext = jnp.concatenate([vu, vb, vd], axis=0)
    zc = jnp.zeros((BH + 2, 1, DIM), jnp.float32)
    v_pad = jnp.concatenate([zc, v_ext, zc], axis=1)
    acc = jnp.broadcast_to(lepe_b.reshape(1, 1, DIM), (BH, H, DIM))
    for ky in range(3):
        for kx in range(3):
            acc = acc + w9[ky, kx] * v_pad[ky:ky + BH, kx:kx + H, :]
    return acc


def _final_body(a_ref, x_ref, xu_ref, xd_ref,
                wv_ref, w9_ref, lb_ref, wo_ref, wob_ref, o_ref):
    i = pl.program_id(0)
    lepe = _lepe_band(i, x_ref, xu_ref, xd_ref, wv_ref[...], w9_ref[...],
                      lb_ref[...])
    s = a_ref[0].reshape(BH * H, DIM) + lepe.reshape(BH * H, DIM)
    o = jnp.dot(s, wo_ref[...], preferred_element_type=jnp.float32) + wob_ref[...]
    o_ref[0] = o.reshape(BH, H, DIM)


def _final(a_img, x, wv, w9, lepe_b, wo_w, wo_b):
    f32 = jnp.float32
    up = lambda i: (0, jnp.maximum(i * BH - 1, 0), 0, 0)
    dn = lambda i: (0, jnp.minimum((i + 1) * BH, H - 1), 0, 0)
    outs = jax.ShapeDtypeStruct((BANDS, BH, H, DIM), f32)
    bspec = pl.BlockSpec((1, BH, H, DIM), lambda i: (i, 0, 0, 0))
    xspec = pl.BlockSpec((1, BH, H, DIM), lambda i: (0, i, 0, 0))
    rspec_u = pl.BlockSpec((1, 1, H, DIM), up)
    rspec_d = pl.BlockSpec((1, 1, H, DIM), dn)
    full2 = lambda i: (0, 0)
    in_specs = [
        bspec,
        xspec, rspec_u, rspec_d,
        pl.BlockSpec((DIM, DIM), full2),
        pl.BlockSpec((3, 3, DIM), lambda i: (0, 0, 0)),
        pl.BlockSpec((1, DIM), full2),
        pl.BlockSpec((DIM, DIM), full2),
        pl.BlockSpec((1, DIM), full2),
    ]
    return pl.pallas_call(
        _final_body,
        grid=(BANDS,),
        in_specs=in_specs,
        out_specs=bspec,
        out_shape=outs,
        interpret=_INTERPRET,
    )(a_img.reshape(BANDS, BH, H, DIM), x, x, x, wv, w9,
      lepe_b.reshape(1, -1), wo_w, wo_b.reshape(1, -1))


def kernel(x1, x2, qkv_w, qkv_b, wo_w, wo_b, lepe_w, lepe_b):
    pmat = _pool_matrix()
    q1, q2, kvp1, kvp2, qm1, km1, qm2, km2 = _stage_a(
        x1, x2, qkv_w, qkv_b, pmat)
    idx1, idx2 = _router(qm1.reshape(P2, QK), km1.reshape(P2, QK),
                         qm2.reshape(P2, QK), km2.reshape(P2, QK))
    a1, a2 = _attention(q1, q2, kvp1, kvp2, idx1, idx2)
    wv = qkv_w[:, 2 * QK:]
    w9 = lepe_w[:, 0].transpose(1, 2, 0)
    o1 = _final(a1.reshape(BANDS, BH, H, DIM), x1, wv, w9, lepe_b, wo_w, wo_b)
    o2 = _final(a2.reshape(BANDS, BH, H, DIM), x2, wv, w9, lepe_b, wo_w, wo_b)
    return o1.reshape(1, H, H, DIM), o2.reshape(1, H, H, DIM)


# speedup vs baseline: 2.3633x; 1.2536x over previous
import functools

import numpy as np
import jax
import jax.numpy as jnp
from jax import lax
from jax.experimental import pallas as pl
from jax.experimental.pallas import tpu as pltpu

DIM = 192
QK = 192
NWIN = 8
P2 = NWIN * NWIN
WS = 28
W2 = WS * WS
NH = 8
HD = DIM // NH
TOPK = 4
KVW = 4
NKV = KVW * KVW
SCALE = QK ** (-0.5)
H = 224
BH = 14
BANDS = H // BH

_INTERPRET = False


def _pool_matrix():
    p = np.zeros((NKV, W2), np.float32)
    for a in range(KVW):
        for b in range(KVW):
            for r in range(7 * a, 7 * a + 7):
                for c in range(7 * b, 7 * b + 7):
                    p[a * KVW + b, r * WS + c] = 1.0 / 49.0
    return jnp.asarray(p)


def _stage_a_body(x1_ref, x2_ref, w_ref, b_ref, p_ref,
                  q1_ref, q2_ref, kvp1_ref, kvp2_ref,
                  qm1_ref, km1_ref, qm2_ref, km2_ref):
    w = w_ref[...]
    b = b_ref[...]
    pmat = p_ref[...]
    for x_ref, q_ref, kvp_ref, qm_ref, km_ref in (
            (x1_ref, q1_ref, kvp1_ref, qm1_ref, km1_ref),
            (x2_ref, q2_ref, kvp2_ref, qm2_ref, km2_ref)):
        band = x_ref[0]
        qs, kvps, qms, kms = [], [], [], []
        for c in range(NWIN):
            xw = band[:, c * WS:(c + 1) * WS, :].reshape(W2, DIM)
            t = jnp.dot(xw, w, preferred_element_type=jnp.float32) + b
            qs.append(t[:, :QK])
            kvps.append(jnp.dot(pmat, t[:, QK:],
                                preferred_element_type=jnp.float32))
            m = jnp.mean(t, axis=0, keepdims=True)
            qms.append(m[:, :QK])
            kms.append(m[:, QK:2 * QK])
        q_ref[...] = jnp.stack(qs)
        kvp_ref[...] = jnp.stack(kvps)
        qm_ref[...] = jnp.stack(qms)
        km_ref[...] = jnp.stack(kms)


def _stage_a(x1, x2, qkv_w, qkv_b, pmat):
    f32 = jnp.float32
    outs = (
        jax.ShapeDtypeStruct((P2, W2, QK), f32),
        jax.ShapeDtypeStruct((P2, W2, QK), f32),
        jax.ShapeDtypeStruct((P2, NKV, 2 * QK), f32),
        jax.ShapeDtypeStruct((P2, NKV, 2 * QK), f32),
        jax.ShapeDtypeStruct((P2, 1, QK), f32),
        jax.ShapeDtypeStruct((P2, 1, QK), f32),
        jax.ShapeDtypeStruct((P2, 1, QK), f32),
        jax.ShapeDtypeStruct((P2, 1, QK), f32),
    )
    band = lambda i: (i, 0, 0)
    xband = lambda i: (0, i, 0, 0)
    full2 = lambda i: (0, 0)
    in_specs = [
        pl.BlockSpec((1, WS, H, DIM), xband),
        pl.BlockSpec((1, WS, H, DIM), xband),
        pl.BlockSpec((DIM, 2 * QK + DIM), full2),
        pl.BlockSpec((1, 2 * QK + DIM), full2),
        pl.BlockSpec((NKV, W2), full2),
    ]
    out_specs = (
        pl.BlockSpec((NWIN, W2, QK), band),
        pl.BlockSpec((NWIN, W2, QK), band),
        pl.BlockSpec((NWIN, NKV, 2 * QK), band),
        pl.BlockSpec((NWIN, NKV, 2 * QK), band),
        pl.BlockSpec((NWIN, 1, QK), band),
        pl.BlockSpec((NWIN, 1, QK), band),
        pl.BlockSpec((NWIN, 1, QK), band),
        pl.BlockSpec((NWIN, 1, QK), band),
    )
    return pl.pallas_call(
        _stage_a_body,
        grid=(NWIN,),
        in_specs=in_specs,
        out_specs=out_specs,
        out_shape=outs,
        interpret=_INTERPRET,
    )(x1, x2, qkv_w, qkv_b.reshape(1, -1), pmat)


def _topk_rows(logits):
    colid = lax.broadcasted_iota(jnp.int32, (P2, P2), 1)
    idxs = []
    for _ in range(TOPK):
        mx = jnp.max(logits, axis=1, keepdims=True)
        cand = jnp.where(logits >= mx, colid, P2)
        am = jnp.min(cand, axis=1, keepdims=True)
        idxs.append(am)
        logits = jnp.where(colid == am, -jnp.float32(np.inf), logits)
    return jnp.concatenate(idxs, axis=1)


def _router_body(qm1_ref, km1_ref, qm2_ref, km2_ref, idx1_ref, idx2_ref):
    rowid = lax.broadcasted_iota(jnp.int32, (P2, P2), 0)
    colid = lax.broadcasted_iota(jnp.int32, (P2, P2), 1)
    diag = rowid == colid
    dn = (((1,), (1,)), ((), ()))
    l1 = lax.dot_general(qm2_ref[...] * SCALE, km1_ref[...], dn,
                         preferred_element_type=jnp.float32)
    l1 = jnp.where(diag, 1.0, l1)
    idx1_ref[...] = _topk_rows(l1)
    l2 = lax.dot_general(qm1_ref[...] * SCALE, km2_ref[...], dn,
                         preferred_element_type=jnp.float32)
    l2 = jnp.where(diag, 1.0, l2)
    idx2_ref[...] = _topk_rows(l2)


def _router(qm1, km1, qm2, km2):
    outs = (jax.ShapeDtypeStruct((P2, TOPK), jnp.int32),
            jax.ShapeDtypeStruct((P2, TOPK), jnp.int32))
    spec = pl.BlockSpec((P2, QK), lambda: (0, 0))
    ospec = pl.BlockSpec((P2, TOPK), lambda: (0, 0))
    return pl.pallas_call(
        _router_body,
        grid=(),
        in_specs=[spec] * 4,
        out_specs=(ospec, ospec),
        out_shape=outs,
        interpret=_INTERPRET,
    )(qm1, km1, qm2, km2)


def _attend_one(q_ref, kvp_ref, idx_ref, wi, wslot):
    rows = []
    for j in range(TOPK):
        r = idx_ref[wi, j]
        rows.append(kvp_ref[r])
    kv = jnp.concatenate(rows, axis=0)
    k_sel = kv[:, :QK]
    v_sel = kv[:, QK:]
    q = q_ref[wslot] * SCALE
    dn = (((1,), (1,)), ((), ()))
    outs = []
    for h in range(NH):
        qh = q[:, h * HD:(h + 1) * HD]
        kh = k_sel[:, h * HD:(h + 1) * HD]
        vh = v_sel[:, h * HD:(h + 1) * HD]
        lg = lax.dot_general(qh, kh, dn, preferred_element_type=jnp.float32)
        m = jnp.max(lg, axis=1, keepdims=True)
        e = jnp.exp(lg - m)
        p = e / jnp.sum(e, axis=1, keepdims=True)
        outs.append(jnp.dot(p, vh, preferred_element_type=jnp.float32))
    return jnp.concatenate(outs, axis=1).reshape(WS, WS, DIM)


def _attn_body(idx1_ref, idx2_ref, q1_ref, q2_ref, kvp1_ref, kvp2_ref,
               a1_ref, a2_ref):
    i = pl.program_id(0)
    for q_ref, kvp_ref, idx_ref, a_ref in (
            (q2_ref, kvp1_ref, idx1_ref, a1_ref),
            (q1_ref, kvp2_ref, idx2_ref, a2_ref)):
        wins = [_attend_one(q_ref, kvp_ref, idx_ref, i * NWIN + c, c)
                for c in range(NWIN)]
        a_ref[0] = jnp.concatenate(wins, axis=1)


def _attention(q1, q2, kvp1, kvp2, idx1, idx2):
    f32 = jnp.float32
    band = lambda i: (i, 0, 0)
    aband = lambda i: (0, i, 0, 0)
    full3 = lambda i: (0, 0, 0)
    outs = (jax.ShapeDtypeStruct((1, H, H, DIM), f32),
            jax.ShapeDtypeStruct((1, H, H, DIM), f32))
    in_specs = [
        pl.BlockSpec(memory_space=pltpu.SMEM),
        pl.BlockSpec(memory_space=pltpu.SMEM),
        pl.BlockSpec((NWIN, W2, QK), band),
        pl.BlockSpec((NWIN, W2, QK), band),
        pl.BlockSpec((P2, NKV, 2 * QK), full3),
        pl.BlockSpec((P2, NKV, 2 * QK), full3),
    ]
    out_specs = (pl.BlockSpec((1, WS, H, DIM), aband),
                 pl.BlockSpec((1, WS, H, DIM), aband))
    return pl.pallas_call(
        _attn_body,
        grid=(NWIN,),
        in_specs=in_specs,
        out_specs=out_specs,
        out_shape=outs,
        interpret=_INTERPRET,
    )(idx1, idx2, q1, q2, kvp1, kvp2)


def _lepe_band(i, x_ref, xu_ref, xd_ref, wv, w9, lepe_b):
    vb = jnp.dot(x_ref[0].reshape(BH * H, DIM), wv,
                 preferred_element_type=jnp.float32).reshape(BH, H, DIM)
    vu = jnp.dot(xu_ref[0, 0], wv, preferred_element_type=jnp.float32)
    vu = jnp.where(i == 0, 0.0, vu)[None]
    vd = jnp.dot(xd_ref[0, 0], wv, preferred_element_type=jnp.float32)
    vd = jnp.where(i == BANDS - 1, 0.0, vd)[None]
    v_ext = jnp.concatenate([vu, vb, vd], axis=0)
    zc = jnp.zeros((BH + 2, 1, DIM), jnp.float32)
    v_pad = jnp.concatenate([zc, v_ext, zc], axis=1)
    acc = jnp.broadcast_to(lepe_b.reshape(1, 1, DIM), (BH, H, DIM))
    for ky in range(3):
        for kx in range(3):
            acc = acc + w9[ky, kx] * v_pad[ky:ky + BH, kx:kx + H, :]
    return acc


def _final_body(a_ref, x_ref, xu_ref, xd_ref,
                wv_ref, w9_ref, lb_ref, wo_ref, wob_ref, o_ref):
    i = pl.program_id(0)
    lepe = _lepe_band(i, x_ref, xu_ref, xd_ref, wv_ref[...], w9_ref[...],
                      lb_ref[...])
    s = a_ref[0].reshape(BH * H, DIM) + lepe.reshape(BH * H, DIM)
    o = jnp.dot(s, wo_ref[...], preferred_element_type=jnp.float32) + wob_ref[...]
    o_ref[0] = jnp.transpose(o.reshape(BH, H, DIM), (0, 2, 1))


def _final(a_img, x, wv, w9, lepe_b, wo_w, wo_b):
    f32 = jnp.float32
    up = lambda i: (0, jnp.maximum(i * BH - 1, 0), 0, 0)
    dn = lambda i: (0, jnp.minimum((i + 1) * BH, H - 1), 0, 0)
    outs = jax.ShapeDtypeStruct((BANDS, BH, DIM, H), f32)
    bspec = pl.BlockSpec((1, BH, H, DIM), lambda i: (i, 0, 0, 0))
    ospec = pl.BlockSpec((1, BH, DIM, H), lambda i: (i, 0, 0, 0))
    xspec = pl.BlockSpec((1, BH, H, DIM), lambda i: (0, i, 0, 0))
    rspec_u = pl.BlockSpec((1, 1, H, DIM), up)
    rspec_d = pl.BlockSpec((1, 1, H, DIM), dn)
    full2 = lambda i: (0, 0)
    in_specs = [
        bspec,
        xspec, rspec_u, rspec_d,
        pl.BlockSpec((DIM, DIM), full2),
        pl.BlockSpec((3, 3, DIM), lambda i: (0, 0, 0)),
        pl.BlockSpec((1, DIM), full2),
        pl.BlockSpec((DIM, DIM), full2),
        pl.BlockSpec((1, DIM), full2),
    ]
    return pl.pallas_call(
        _final_body,
        grid=(BANDS,),
        in_specs=in_specs,
        out_specs=ospec,
        out_shape=outs,
        interpret=_INTERPRET,
    )(a_img.reshape(BANDS, BH, H, DIM), x, x, x, wv, w9,
      lepe_b.reshape(1, -1), wo_w, wo_b.reshape(1, -1))


def kernel(x1, x2, qkv_w, qkv_b, wo_w, wo_b, lepe_w, lepe_b):
    pmat = _pool_matrix()
    q1, q2, kvp1, kvp2, qm1, km1, qm2, km2 = _stage_a(
        x1, x2, qkv_w, qkv_b, pmat)
    idx1, idx2 = _router(qm1.reshape(P2, QK), km1.reshape(P2, QK),
                         qm2.reshape(P2, QK), km2.reshape(P2, QK))
    a1, a2 = _attention(q1, q2, kvp1, kvp2, idx1, idx2)
    wv = qkv_w[:, 2 * QK:]
    w9 = lepe_w[:, 0].transpose(1, 2, 0)
    o1 = _final(a1.reshape(BANDS, BH, H, DIM), x1, wv, w9, lepe_b, wo_w, wo_b)
    o2 = _final(a2.reshape(BANDS, BH, H, DIM), x2, wv, w9, lepe_b, wo_w, wo_b)
    o1 = jnp.transpose(o1.reshape(1, H, DIM, H), (0, 1, 3, 2))
    o2 = jnp.transpose(o2.reshape(1, H, DIM, H), (0, 1, 3, 2))
    return o1, o2


# speedup vs baseline: 2.5137x; 1.0636x over previous
import functools

import numpy as np
import jax
import jax.numpy as jnp
from jax import lax
from jax.experimental import pallas as pl
from jax.experimental.pallas import tpu as pltpu

DIM = 192
QK = 192
NWIN = 8
P2 = NWIN * NWIN
WS = 28
W2 = WS * WS
NH = 8
HD = DIM // NH
TOPK = 4
KVW = 4
NKV = KVW * KVW
SCALE = QK ** (-0.5)
H = 224
BH = 14
BANDS = H // BH

_INTERPRET = False


def _pool_matrix():
    p = np.zeros((NKV, W2), np.float32)
    for a in range(KVW):
        for b in range(KVW):
            for r in range(7 * a, 7 * a + 7):
                for c in range(7 * b, 7 * b + 7):
                    p[a * KVW + b, r * WS + c] = 1.0 / 49.0
    return jnp.asarray(p)


def _stage_a_body(x1_ref, x2_ref, w_ref, b_ref, p_ref,
                  q1_ref, q2_ref, kvp1_ref, kvp2_ref,
                  qm1_ref, km1_ref, qm2_ref, km2_ref):
    w = w_ref[...]
    b = b_ref[...]
    pmat = p_ref[...]
    for x_ref, q_ref, kvp_ref, qm_ref, km_ref in (
            (x1_ref, q1_ref, kvp1_ref, qm1_ref, km1_ref),
            (x2_ref, q2_ref, kvp2_ref, qm2_ref, km2_ref)):
        band = jnp.transpose(x_ref[0], (0, 2, 1))
        qs, kvps, qms, kms = [], [], [], []
        for c in range(NWIN):
            xw = band[:, c * WS:(c + 1) * WS, :].reshape(W2, DIM)
            t = jnp.dot(xw, w, preferred_element_type=jnp.float32) + b
            qs.append(t[:, :QK])
            kvps.append(jnp.dot(pmat, t[:, QK:],
                                preferred_element_type=jnp.float32))
            m = jnp.mean(t, axis=0, keepdims=True)
            qms.append(m[:, :QK])
            kms.append(m[:, QK:2 * QK])
        q_ref[...] = jnp.stack(qs)
        kvp_ref[...] = jnp.stack(kvps)
        qm_ref[...] = jnp.stack(qms)
        km_ref[...] = jnp.stack(kms)


def _stage_a(x1, x2, qkv_w, qkv_b, pmat):
    f32 = jnp.float32
    outs = (
        jax.ShapeDtypeStruct((P2, W2, QK), f32),
        jax.ShapeDtypeStruct((P2, W2, QK), f32),
        jax.ShapeDtypeStruct((P2, NKV, 2 * QK), f32),
        jax.ShapeDtypeStruct((P2, NKV, 2 * QK), f32),
        jax.ShapeDtypeStruct((P2, 1, QK), f32),
        jax.ShapeDtypeStruct((P2, 1, QK), f32),
        jax.ShapeDtypeStruct((P2, 1, QK), f32),
        jax.ShapeDtypeStruct((P2, 1, QK), f32),
    )
    band = lambda i: (i, 0, 0)
    xband = lambda i: (0, i, 0, 0)
    full2 = lambda i: (0, 0)
    in_specs = [
        pl.BlockSpec((1, WS, DIM, H), xband),
        pl.BlockSpec((1, WS, DIM, H), xband),
        pl.BlockSpec((DIM, 2 * QK + DIM), full2),
        pl.BlockSpec((1, 2 * QK + DIM), full2),
        pl.BlockSpec((NKV, W2), full2),
    ]
    out_specs = (
        pl.BlockSpec((NWIN, W2, QK), band),
        pl.BlockSpec((NWIN, W2, QK), band),
        pl.BlockSpec((NWIN, NKV, 2 * QK), band),
        pl.BlockSpec((NWIN, NKV, 2 * QK), band),
        pl.BlockSpec((NWIN, 1, QK), band),
        pl.BlockSpec((NWIN, 1, QK), band),
        pl.BlockSpec((NWIN, 1, QK), band),
        pl.BlockSpec((NWIN, 1, QK), band),
    )
    return pl.pallas_call(
        _stage_a_body,
        grid=(NWIN,),
        in_specs=in_specs,
        out_specs=out_specs,
        out_shape=outs,
        interpret=_INTERPRET,
    )(x1, x2, qkv_w, qkv_b.reshape(1, -1), pmat)


def _topk_rows(logits):
    colid = lax.broadcasted_iota(jnp.int32, (P2, P2), 1)
    idxs = []
    for _ in range(TOPK):
        mx = jnp.max(logits, axis=1, keepdims=True)
        cand = jnp.where(logits >= mx, colid, P2)
        am = jnp.min(cand, axis=1, keepdims=True)
        idxs.append(am)
        logits = jnp.where(colid == am, -jnp.float32(np.inf), logits)
    return jnp.concatenate(idxs, axis=1)


def _router_body(qm1_ref, km1_ref, qm2_ref, km2_ref, idx1_ref, idx2_ref):
    rowid = lax.broadcasted_iota(jnp.int32, (P2, P2), 0)
    colid = lax.broadcasted_iota(jnp.int32, (P2, P2), 1)
    diag = rowid == colid
    dn = (((1,), (1,)), ((), ()))
    l1 = lax.dot_general(qm2_ref[...] * SCALE, km1_ref[...], dn,
                         preferred_element_type=jnp.float32)
    l1 = jnp.where(diag, 1.0, l1)
    idx1_ref[...] = _topk_rows(l1)
    l2 = lax.dot_general(qm1_ref[...] * SCALE, km2_ref[...], dn,
                         preferred_element_type=jnp.float32)
    l2 = jnp.where(diag, 1.0, l2)
    idx2_ref[...] = _topk_rows(l2)


def _router(qm1, km1, qm2, km2):
    outs = (jax.ShapeDtypeStruct((P2, TOPK), jnp.int32),
            jax.ShapeDtypeStruct((P2, TOPK), jnp.int32))
    spec = pl.BlockSpec((P2, QK), lambda: (0, 0))
    ospec = pl.BlockSpec((P2, TOPK), lambda: (0, 0))
    return pl.pallas_call(
        _router_body,
        grid=(),
        in_specs=[spec] * 4,
        out_specs=(ospec, ospec),
        out_shape=outs,
        interpret=_INTERPRET,
    )(qm1, km1, qm2, km2)


def _attend_one(q_ref, kvp_ref, idx_ref, wi, wslot):
    rows = []
    for j in range(TOPK):
        r = idx_ref[wi, j]
        rows.append(kvp_ref[r])
    kv = jnp.concatenate(rows, axis=0)
    k_sel = kv[:, :QK]
    v_sel = kv[:, QK:]
    q = q_ref[wslot] * SCALE
    dn = (((1,), (1,)), ((), ()))
    outs = []
    for h in range(NH):
        qh = q[:, h * HD:(h + 1) * HD]
        kh = k_sel[:, h * HD:(h + 1) * HD]
        vh = v_sel[:, h * HD:(h + 1) * HD]
        lg = lax.dot_general(qh, kh, dn, preferred_element_type=jnp.float32)
        m = jnp.max(lg, axis=1, keepdims=True)
        e = jnp.exp(lg - m)
        p = e / jnp.sum(e, axis=1, keepdims=True)
        outs.append(jnp.dot(p, vh, preferred_element_type=jnp.float32))
    return jnp.concatenate(outs, axis=1).reshape(WS, WS, DIM)


def _attn_body(idx1_ref, idx2_ref, q1_ref, q2_ref, kvp1_ref, kvp2_ref,
               a1_ref, a2_ref):
    i = pl.program_id(0)
    for q_ref, kvp_ref, idx_ref, a_ref in (
            (q2_ref, kvp1_ref, idx1_ref, a1_ref),
            (q1_ref, kvp2_ref, idx2_ref, a2_ref)):
        wins = [_attend_one(q_ref, kvp_ref, idx_ref, i * NWIN + c, c)
                for c in range(NWIN)]
        a_ref[0] = jnp.concatenate(wins, axis=1)


def _attention(q1, q2, kvp1, kvp2, idx1, idx2):
    f32 = jnp.float32
    band = lambda i: (i, 0, 0)
    aband = lambda i: (0, i, 0, 0)
    full3 = lambda i: (0, 0, 0)
    outs = (jax.ShapeDtypeStruct((1, H, H, DIM), f32),
            jax.ShapeDtypeStruct((1, H, H, DIM), f32))
    in_specs = [
        pl.BlockSpec(memory_space=pltpu.SMEM),
        pl.BlockSpec(memory_space=pltpu.SMEM),
        pl.BlockSpec((NWIN, W2, QK), band),
        pl.BlockSpec((NWIN, W2, QK), band),
        pl.BlockSpec((P2, NKV, 2 * QK), full3),
        pl.BlockSpec((P2, NKV, 2 * QK), full3),
    ]
    out_specs = (pl.BlockSpec((1, WS, H, DIM), aband),
                 pl.BlockSpec((1, WS, H, DIM), aband))
    return pl.pallas_call(
        _attn_body,
        grid=(NWIN,),
        in_specs=in_specs,
        out_specs=out_specs,
        out_shape=outs,
        interpret=_INTERPRET,
    )(idx1, idx2, q1, q2, kvp1, kvp2)


def _lepe_band(i, x_ref, xu_ref, xd_ref, wv, w9, lepe_b):
    xb = jnp.transpose(x_ref[0], (0, 2, 1))
    vb = jnp.dot(xb.reshape(BH * H, DIM), wv,
                 preferred_element_type=jnp.float32).reshape(BH, H, DIM)
    xu = jnp.transpose(xu_ref[0, 0], (1, 0))
    vu = jnp.dot(xu, wv, preferred_element_type=jnp.float32)
    vu = jnp.where(i == 0, 0.0, vu)[None]
    xd = jnp.transpose(xd_ref[0, 0], (1, 0))
    vd = jnp.dot(xd, wv, preferred_element_type=jnp.float32)
    vd = jnp.where(i == BANDS - 1, 0.0, vd)[None]
    v_ext = jnp.concatenate([vu, vb, vd], axis=0)
    zc = jnp.zeros((BH + 2, 1, DIM), jnp.float32)
    v_pad = jnp.concatenate([zc, v_ext, zc], axis=1)
    acc = jnp.broadcast_to(lepe_b.reshape(1, 1, DIM), (BH, H, DIM))
    for ky in range(3):
        for kx in range(3):
            acc = acc + w9[ky, kx] * v_pad[ky:ky + BH, kx:kx + H, :]
    return acc


def _final_body(a_ref, x_ref, xu_ref, xd_ref,
                wv_ref, w9_ref, lb_ref, wo_ref, wob_ref, o_ref):
    i = pl.program_id(0)
    lepe = _lepe_band(i, x_ref, xu_ref, xd_ref, wv_ref[...], w9_ref[...],
                      lb_ref[...])
    s = a_ref[0].reshape(BH * H, DIM) + lepe.reshape(BH * H, DIM)
    o = jnp.dot(s, wo_ref[...], preferred_element_type=jnp.float32) + wob_ref[...]
    o_ref[0] = jnp.transpose(o.reshape(BH, H, DIM), (0, 2, 1))


def _final(a_img, x, wv, w9, lepe_b, wo_w, wo_b):
    f32 = jnp.float32
    up = lambda i: (0, jnp.maximum(i * BH - 1, 0), 0, 0)
    dn = lambda i: (0, jnp.minimum((i + 1) * BH, H - 1), 0, 0)
    outs = jax.ShapeDtypeStruct((BANDS, BH, DIM, H), f32)
    bspec = pl.BlockSpec((1, BH, H, DIM), lambda i: (i, 0, 0, 0))
    ospec = pl.BlockSpec((1, BH, DIM, H), lambda i: (i, 0, 0, 0))
    xspec = pl.BlockSpec((1, BH, DIM, H), lambda i: (0, i, 0, 0))
    rspec_u = pl.BlockSpec((1, 1, DIM, H), up)
    rspec_d = pl.BlockSpec((1, 1, DIM, H), dn)
    full2 = lambda i: (0, 0)
    in_specs = [
        bspec,
        xspec, rspec_u, rspec_d,
        pl.BlockSpec((DIM, DIM), full2),
        pl.BlockSpec((3, 3, DIM), lambda i: (0, 0, 0)),
        pl.BlockSpec((1, DIM), full2),
        pl.BlockSpec((DIM, DIM), full2),
        pl.BlockSpec((1, DIM), full2),
    ]
    return pl.pallas_call(
        _final_body,
        grid=(BANDS,),
        in_specs=in_specs,
        out_specs=ospec,
        out_shape=outs,
        interpret=_INTERPRET,
    )(a_img.reshape(BANDS, BH, H, DIM), x, x, x, wv, w9,
      lepe_b.reshape(1, -1), wo_w, wo_b.reshape(1, -1))


def kernel(x1, x2, qkv_w, qkv_b, wo_w, wo_b, lepe_w, lepe_b):
    pmat = _pool_matrix()
    x1t = jnp.transpose(x1, (0, 1, 3, 2))
    x2t = jnp.transpose(x2, (0, 1, 3, 2))
    q1, q2, kvp1, kvp2, qm1, km1, qm2, km2 = _stage_a(
        x1t, x2t, qkv_w, qkv_b, pmat)
    idx1, idx2 = _router(qm1.reshape(P2, QK), km1.reshape(P2, QK),
                         qm2.reshape(P2, QK), km2.reshape(P2, QK))
    a1, a2 = _attention(q1, q2, kvp1, kvp2, idx1, idx2)
    wv = qkv_w[:, 2 * QK:]
    w9 = lepe_w[:, 0].transpose(1, 2, 0)
    o1 = _final(a1.reshape(BANDS, BH, H, DIM), x1t, wv, w9, lepe_b, wo_w, wo_b)
    o2 = _final(a2.reshape(BANDS, BH, H, DIM), x2t, wv, w9, lepe_b, wo_w, wo_b)
    o1 = jnp.transpose(o1.reshape(1, H, DIM, H), (0, 1, 3, 2))
    o2 = jnp.transpose(o2.reshape(1, H, DIM, H), (0, 1, 3, 2))
    return o1, o2


# speedup vs baseline: 2.5166x; 1.0011x over previous
import functools

import numpy as np
import jax
import jax.numpy as jnp
from jax import lax
from jax.experimental import pallas as pl
from jax.experimental.pallas import tpu as pltpu

DIM = 192
QK = 192
NWIN = 8
P2 = NWIN * NWIN
WS = 28
W2 = WS * WS
NH = 8
HD = DIM // NH
TOPK = 4
KVW = 4
NKV = KVW * KVW
SCALE = QK ** (-0.5)
H = 224
BH = 14
BANDS = H // BH

_INTERPRET = False


def _pool_matrix():
    p = np.zeros((NKV, W2), np.float32)
    for a in range(KVW):
        for b in range(KVW):
            for r in range(7 * a, 7 * a + 7):
                for c in range(7 * b, 7 * b + 7):
                    p[a * KVW + b, r * WS + c] = 1.0 / 49.0
    return jnp.asarray(p)


def _stage_a_body(x1_ref, x2_ref, w_ref, b_ref, p_ref,
                  q1_ref, q2_ref, kvp1_ref, kvp2_ref,
                  qm1_ref, km1_ref, qm2_ref, km2_ref):
    wf = w_ref[...]
    w = wf.astype(jnp.bfloat16)
    b = b_ref[...]
    pmat = p_ref[...]
    for x_ref, q_ref, kvp_ref, qm_ref, km_ref in (
            (x1_ref, q1_ref, kvp1_ref, qm1_ref, km1_ref),
            (x2_ref, q2_ref, kvp2_ref, qm2_ref, km2_ref)):
        bandf = jnp.transpose(x_ref[0], (0, 2, 1))
        band = bandf.astype(jnp.bfloat16)
        qs, kvps, qms, kms = [], [], [], []
        for c in range(NWIN):
            xw = band[:, c * WS:(c + 1) * WS, :].reshape(W2, DIM)
            t = jnp.dot(xw, w, preferred_element_type=jnp.float32) + b
            qs.append(t[:, :QK])
            kvps.append(jnp.dot(pmat, t[:, QK:],
                                preferred_element_type=jnp.float32))
            xwf = bandf[:, c * WS:(c + 1) * WS, :].reshape(W2, DIM)
            xmean = jnp.mean(xwf, axis=0, keepdims=True)
            m = jnp.dot(xmean, wf[:, :2 * QK],
                        preferred_element_type=jnp.float32) + b[:, :2 * QK]
            qms.append(m[:, :QK])
            kms.append(m[:, QK:2 * QK])
        q_ref[...] = jnp.stack(qs)
        kvp_ref[...] = jnp.stack(kvps)
        qm_ref[...] = jnp.stack(qms)
        km_ref[...] = jnp.stack(kms)


def _stage_a(x1, x2, qkv_w, qkv_b, pmat):
    f32 = jnp.float32
    outs = (
        jax.ShapeDtypeStruct((P2, W2, QK), f32),
        jax.ShapeDtypeStruct((P2, W2, QK), f32),
        jax.ShapeDtypeStruct((P2, NKV, 2 * QK), f32),
        jax.ShapeDtypeStruct((P2, NKV, 2 * QK), f32),
        jax.ShapeDtypeStruct((P2, 1, QK), f32),
        jax.ShapeDtypeStruct((P2, 1, QK), f32),
        jax.ShapeDtypeStruct((P2, 1, QK), f32),
        jax.ShapeDtypeStruct((P2, 1, QK), f32),
    )
    band = lambda i: (i, 0, 0)
    xband = lambda i: (0, i, 0, 0)
    full2 = lambda i: (0, 0)
    in_specs = [
        pl.BlockSpec((1, WS, DIM, H), xband),
        pl.BlockSpec((1, WS, DIM, H), xband),
        pl.BlockSpec((DIM, 2 * QK + DIM), full2),
        pl.BlockSpec((1, 2 * QK + DIM), full2),
        pl.BlockSpec((NKV, W2), full2),
    ]
    out_specs = (
        pl.BlockSpec((NWIN, W2, QK), band),
        pl.BlockSpec((NWIN, W2, QK), band),
        pl.BlockSpec((NWIN, NKV, 2 * QK), band),
        pl.BlockSpec((NWIN, NKV, 2 * QK), band),
        pl.BlockSpec((NWIN, 1, QK), band),
        pl.BlockSpec((NWIN, 1, QK), band),
        pl.BlockSpec((NWIN, 1, QK), band),
        pl.BlockSpec((NWIN, 1, QK), band),
    )
    return pl.pallas_call(
        _stage_a_body,
        grid=(NWIN,),
        in_specs=in_specs,
        out_specs=out_specs,
        out_shape=outs,
        interpret=_INTERPRET,
    )(x1, x2, qkv_w, qkv_b.reshape(1, -1), pmat)


def _topk_rows(logits):
    colid = lax.broadcasted_iota(jnp.int32, (P2, P2), 1)
    idxs = []
    for _ in range(TOPK):
        mx = jnp.max(logits, axis=1, keepdims=True)
        cand = jnp.where(logits >= mx, colid, P2)
        am = jnp.min(cand, axis=1, keepdims=True)
        idxs.append(am)
        logits = jnp.where(colid == am, -jnp.float32(np.inf), logits)
    return jnp.concatenate(idxs, axis=1)


def _router_body(qm1_ref, km1_ref, qm2_ref, km2_ref, idx1_ref, idx2_ref):
    rowid = lax.broadcasted_iota(jnp.int32, (P2, P2), 0)
    colid = lax.broadcasted_iota(jnp.int32, (P2, P2), 1)
    diag = rowid == colid
    dn = (((1,), (1,)), ((), ()))
    l1 = lax.dot_general(qm2_ref[...] * SCALE, km1_ref[...], dn,
                         preferred_element_type=jnp.float32)
    l1 = jnp.where(diag, 1.0, l1)
    idx1_ref[...] = _topk_rows(l1)
    l2 = lax.dot_general(qm1_ref[...] * SCALE, km2_ref[...], dn,
                         preferred_element_type=jnp.float32)
    l2 = jnp.where(diag, 1.0, l2)
    idx2_ref[...] = _topk_rows(l2)


def _router(qm1, km1, qm2, km2):
    outs = (jax.ShapeDtypeStruct((P2, TOPK), jnp.int32),
            jax.ShapeDtypeStruct((P2, TOPK), jnp.int32))
    spec = pl.BlockSpec((P2, QK), lambda: (0, 0))
    ospec = pl.BlockSpec((P2, TOPK), lambda: (0, 0))
    return pl.pallas_call(
        _router_body,
        grid=(),
        in_specs=[spec] * 4,
        out_specs=(ospec, ospec),
        out_shape=outs,
        interpret=_INTERPRET,
    )(qm1, km1, qm2, km2)


def _attend_one(q_ref, kvp_ref, idx_ref, wi, wslot):
    rows = []
    for j in range(TOPK):
        r = idx_ref[wi, j]
        rows.append(kvp_ref[r])
    kv = jnp.concatenate(rows, axis=0)
    k_sel = kv[:, :QK]
    v_sel = kv[:, QK:]
    q = q_ref[wslot] * SCALE
    dn = (((1,), (1,)), ((), ()))
    outs = []
    for h in range(NH):
        qh = q[:, h * HD:(h + 1) * HD]
        kh = k_sel[:, h * HD:(h + 1) * HD]
        vh = v_sel[:, h * HD:(h + 1) * HD]
        lg = lax.dot_general(qh, kh, dn, preferred_element_type=jnp.float32)
        m = jnp.max(lg, axis=1, keepdims=True)
        e = jnp.exp(lg - m)
        p = e / jnp.sum(e, axis=1, keepdims=True)
        outs.append(jnp.dot(p, vh, preferred_element_type=jnp.float32))
    return jnp.concatenate(outs, axis=1).reshape(WS, WS, DIM)


def _attn_body(idx1_ref, idx2_ref, q1_ref, q2_ref, kvp1_ref, kvp2_ref,
               a1_ref, a2_ref):
    i = pl.program_id(0)
    for q_ref, kvp_ref, idx_ref, a_ref in (
            (q2_ref, kvp1_ref, idx1_ref, a1_ref),
            (q1_ref, kvp2_ref, idx2_ref, a2_ref)):
        wins = [_attend_one(q_ref, kvp_ref, idx_ref, i * NWIN + c, c)
                for c in range(NWIN)]
        a_ref[0] = jnp.concatenate(wins, axis=1)


def _attention(q1, q2, kvp1, kvp2, idx1, idx2):
    f32 = jnp.float32
    band = lambda i: (i, 0, 0)
    aband = lambda i: (0, i, 0, 0)
    full3 = lambda i: (0, 0, 0)
    outs = (jax.ShapeDtypeStruct((1, H, H, DIM), f32),
            jax.ShapeDtypeStruct((1, H, H, DIM), f32))
    in_specs = [
        pl.BlockSpec(memory_space=pltpu.SMEM),
        pl.BlockSpec(memory_space=pltpu.SMEM),
        pl.BlockSpec((NWIN, W2, QK), band),
        pl.BlockSpec((NWIN, W2, QK), band),
        pl.BlockSpec((P2, NKV, 2 * QK), full3),
        pl.BlockSpec((P2, NKV, 2 * QK), full3),
    ]
    out_specs = (pl.BlockSpec((1, WS, H, DIM), aband),
                 pl.BlockSpec((1, WS, H, DIM), aband))
    return pl.pallas_call(
        _attn_body,
        grid=(NWIN,),
        in_specs=in_specs,
        out_specs=out_specs,
        out_shape=outs,
        interpret=_INTERPRET,
    )(idx1, idx2, q1, q2, kvp1, kvp2)


def _lepe_band(i, x_ref, xu_ref, xd_ref, wv, w9, lepe_b):
    xb = jnp.transpose(x_ref[0], (0, 2, 1)).astype(jnp.bfloat16)
    vb = jnp.dot(xb.reshape(BH * H, DIM), wv,
                 preferred_element_type=jnp.float32).reshape(BH, H, DIM)
    xu = jnp.transpose(xu_ref[0, 0], (1, 0)).astype(jnp.bfloat16)
    vu = jnp.dot(xu, wv, preferred_element_type=jnp.float32)
    vu = jnp.where(i == 0, 0.0, vu)[None]
    xd = jnp.transpose(xd_ref[0, 0], (1, 0)).astype(jnp.bfloat16)
    vd = jnp.dot(xd, wv, preferred_element_type=jnp.float32)
    vd = jnp.where(i == BANDS - 1, 0.0, vd)[None]
    v_ext = jnp.concatenate([vu, vb, vd], axis=0)
    zc = jnp.zeros((BH + 2, 1, DIM), jnp.float32)
    v_pad = jnp.concatenate([zc, v_ext, zc], axis=1)
    acc = jnp.broadcast_to(lepe_b.reshape(1, 1, DIM), (BH, H, DIM))
    for ky in range(3):
        for kx in range(3):
            acc = acc + w9[ky, kx] * v_pad[ky:ky + BH, kx:kx + H, :]
    return acc


def _final_body(a_ref, x_ref, xu_ref, xd_ref,
                wv_ref, w9_ref, lb_ref, wo_ref, wob_ref, o_ref):
    i = pl.program_id(0)
    lepe = _lepe_band(i, x_ref, xu_ref, xd_ref,
                      wv_ref[...].astype(jnp.bfloat16), w9_ref[...],
                      lb_ref[...])
    s = a_ref[0].reshape(BH * H, DIM) + lepe.reshape(BH * H, DIM)
    o = jnp.dot(s.astype(jnp.bfloat16), wo_ref[...].astype(jnp.bfloat16),
                preferred_element_type=jnp.float32) + wob_ref[...]
    o_ref[0] = jnp.transpose(o.reshape(BH, H, DIM), (0, 2, 1))


def _final(a_img, x, wv, w9, lepe_b, wo_w, wo_b):
    f32 = jnp.float32
    up = lambda i: (0, jnp.maximum(i * BH - 1, 0), 0, 0)
    dn = lambda i: (0, jnp.minimum((i + 1) * BH, H - 1), 0, 0)
    outs = jax.ShapeDtypeStruct((BANDS, BH, DIM, H), f32)
    bspec = pl.BlockSpec((1, BH, H, DIM), lambda i: (i, 0, 0, 0))
    ospec = pl.BlockSpec((1, BH, DIM, H), lambda i: (i, 0, 0, 0))
    xspec = pl.BlockSpec((1, BH, DIM, H), lambda i: (0, i, 0, 0))
    rspec_u = pl.BlockSpec((1, 1, DIM, H), up)
    rspec_d = pl.BlockSpec((1, 1, DIM, H), dn)
    full2 = lambda i: (0, 0)
    in_specs = [
        bspec,
        xspec, rspec_u, rspec_d,
        pl.BlockSpec((DIM, DIM), full2),
        pl.BlockSpec((3, 3, DIM), lambda i: (0, 0, 0)),
        pl.BlockSpec((1, DIM), full2),
        pl.BlockSpec((DIM, DIM), full2),
        pl.BlockSpec((1, DIM), full2),
    ]
    return pl.pallas_call(
        _final_body,
        grid=(BANDS,),
        in_specs=in_specs,
        out_specs=ospec,
        out_shape=outs,
        interpret=_INTERPRET,
    )(a_img.reshape(BANDS, BH, H, DIM), x, x, x, wv, w9,
      lepe_b.reshape(1, -1), wo_w, wo_b.reshape(1, -1))


def kernel(x1, x2, qkv_w, qkv_b, wo_w, wo_b, lepe_w, lepe_b):
    pmat = _pool_matrix()
    x1t = jnp.transpose(x1, (0, 1, 3, 2))
    x2t = jnp.transpose(x2, (0, 1, 3, 2))
    q1, q2, kvp1, kvp2, qm1, km1, qm2, km2 = _stage_a(
        x1t, x2t, qkv_w, qkv_b, pmat)
    idx1, idx2 = _router(qm1.reshape(P2, QK), km1.reshape(P2, QK),
                         qm2.reshape(P2, QK), km2.reshape(P2, QK))
    a1, a2 = _attention(q1, q2, kvp1, kvp2, idx1, idx2)
    wv = qkv_w[:, 2 * QK:]
    w9 = lepe_w[:, 0].transpose(1, 2, 0)
    o1 = _final(a1.reshape(BANDS, BH, H, DIM), x1t, wv, w9, lepe_b, wo_w, wo_b)
    o2 = _final(a2.reshape(BANDS, BH, H, DIM), x2t, wv, w9, lepe_b, wo_w, wo_b)
    o1 = jnp.transpose(o1.reshape(1, H, DIM, H), (0, 1, 3, 2))
    o2 = jnp.transpose(o2.reshape(1, H, DIM, H), (0, 1, 3, 2))
    return o1, o2


# speedup vs baseline: 2.5274x; 1.0043x over previous
import functools

import numpy as np
import jax
import jax.numpy as jnp
from jax import lax
from jax.experimental import pallas as pl
from jax.experimental.pallas import tpu as pltpu

DIM = 192
QK = 192
NWIN = 8
P2 = NWIN * NWIN
WS = 28
W2 = WS * WS
NH = 8
HD = DIM // NH
TOPK = 4
KVW = 4
NKV = KVW * KVW
SCALE = QK ** (-0.5)
H = 224
BH = 14
BANDS = H // BH

_INTERPRET = False


def _pool_matrix():
    p = np.zeros((NKV, W2), np.float32)
    for a in range(KVW):
        for b in range(KVW):
            for r in range(7 * a, 7 * a + 7):
                for c in range(7 * b, 7 * b + 7):
                    p[a * KVW + b, r * WS + c] = 1.0 / 49.0
    return jnp.asarray(p)


def _stage_a_body(x1_ref, x2_ref, w_ref, b_ref, p_ref,
                  q1_ref, q2_ref, kvp1_ref, kvp2_ref,
                  qm1_ref, km1_ref, qm2_ref, km2_ref):
    w = w_ref[...]
    b = b_ref[...]
    pmat = p_ref[...]
    for x_ref, q_ref, kvp_ref, qm_ref, km_ref in (
            (x1_ref, q1_ref, kvp1_ref, qm1_ref, km1_ref),
            (x2_ref, q2_ref, kvp2_ref, qm2_ref, km2_ref)):
        band = jnp.transpose(x_ref[0], (0, 2, 1))
        qs, kvps, qms, kms = [], [], [], []
        for c in range(NWIN):
            xw = band[:, c * WS:(c + 1) * WS, :].reshape(W2, DIM)
            t = jnp.dot(xw, w, preferred_element_type=jnp.float32) + b
            qs.append(t[:, :QK].astype(jnp.bfloat16))
            kvps.append(jnp.dot(pmat, t[:, QK:],
                                preferred_element_type=jnp.float32)
                        .astype(jnp.bfloat16))
            m = jnp.mean(t, axis=0, keepdims=True)
            qms.append(m[:, :QK])
            kms.append(m[:, QK:2 * QK])
        q_ref[...] = jnp.stack(qs)
        kvp_ref[...] = jnp.stack(kvps)
        qm_ref[...] = jnp.stack(qms)
        km_ref[...] = jnp.stack(kms)


def _stage_a(x1, x2, qkv_w, qkv_b, pmat):
    f32 = jnp.float32
    bf16 = jnp.bfloat16
    outs = (
        jax.ShapeDtypeStruct((P2, W2, QK), bf16),
        jax.ShapeDtypeStruct((P2, W2, QK), bf16),
        jax.ShapeDtypeStruct((P2, NKV, 2 * QK), bf16),
        jax.ShapeDtypeStruct((P2, NKV, 2 * QK), bf16),
        jax.ShapeDtypeStruct((P2, 1, QK), f32),
        jax.ShapeDtypeStruct((P2, 1, QK), f32),
        jax.ShapeDtypeStruct((P2, 1, QK), f32),
        jax.ShapeDtypeStruct((P2, 1, QK), f32),
    )
    band = lambda i: (i, 0, 0)
    xband = lambda i: (0, i, 0, 0)
    full2 = lambda i: (0, 0)
    in_specs = [
        pl.BlockSpec((1, WS, DIM, H), xband),
        pl.BlockSpec((1, WS, DIM, H), xband),
        pl.BlockSpec((DIM, 2 * QK + DIM), full2),
        pl.BlockSpec((1, 2 * QK + DIM), full2),
        pl.BlockSpec((NKV, W2), full2),
    ]
    out_specs = (
        pl.BlockSpec((NWIN, W2, QK), band),
        pl.BlockSpec((NWIN, W2, QK), band),
        pl.BlockSpec((NWIN, NKV, 2 * QK), band),
        pl.BlockSpec((NWIN, NKV, 2 * QK), band),
        pl.BlockSpec((NWIN, 1, QK), band),
        pl.BlockSpec((NWIN, 1, QK), band),
        pl.BlockSpec((NWIN, 1, QK), band),
        pl.BlockSpec((NWIN, 1, QK), band),
    )
    return pl.pallas_call(
        _stage_a_body,
        grid=(NWIN,),
        in_specs=in_specs,
        out_specs=out_specs,
        out_shape=outs,
        interpret=_INTERPRET,
    )(x1, x2, qkv_w, qkv_b.reshape(1, -1), pmat)


def _topk_rows(logits):
    colid = lax.broadcasted_iota(jnp.int32, (P2, P2), 1)
    idxs = []
    for _ in range(TOPK):
        mx = jnp.max(logits, axis=1, keepdims=True)
        cand = jnp.where(logits >= mx, colid, P2)
        am = jnp.min(cand, axis=1, keepdims=True)
        idxs.append(am)
        logits = jnp.where(colid == am, -jnp.float32(np.inf), logits)
    return jnp.concatenate(idxs, axis=1)


def _router_body(qm1_ref, km1_ref, qm2_ref, km2_ref, idx1_ref, idx2_ref):
    rowid = lax.broadcasted_iota(jnp.int32, (P2, P2), 0)
    colid = lax.broadcasted_iota(jnp.int32, (P2, P2), 1)
    diag = rowid == colid
    dn = (((1,), (1,)), ((), ()))
    l1 = lax.dot_general(qm2_ref[...] * SCALE, km1_ref[...], dn,
                         preferred_element_type=jnp.float32)
    l1 = jnp.where(diag, 1.0, l1)
    idx1_ref[...] = _topk_rows(l1)
    l2 = lax.dot_general(qm1_ref[...] * SCALE, km2_ref[...], dn,
                         preferred_element_type=jnp.float32)
    l2 = jnp.where(diag, 1.0, l2)
    idx2_ref[...] = _topk_rows(l2)


def _router(qm1, km1, qm2, km2):
    outs = (jax.ShapeDtypeStruct((P2, TOPK), jnp.int32),
            jax.ShapeDtypeStruct((P2, TOPK), jnp.int32))
    spec = pl.BlockSpec((P2, QK), lambda: (0, 0))
    ospec = pl.BlockSpec((P2, TOPK), lambda: (0, 0))
    return pl.pallas_call(
        _router_body,
        grid=(),
        in_specs=[spec] * 4,
        out_specs=(ospec, ospec),
        out_shape=outs,
        interpret=_INTERPRET,
    )(qm1, km1, qm2, km2)


def _attend_one(q_ref, kvp_ref, idx_ref, wi, wslot):
    rows = []
    for j in range(TOPK):
        r = idx_ref[wi, j]
        rows.append(kvp_ref[r])
    kv = jnp.concatenate(rows, axis=0).astype(jnp.float32)
    k_sel = kv[:, :QK]
    v_sel = kv[:, QK:]
    q = q_ref[wslot].astype(jnp.float32) * SCALE
    dn = (((1,), (1,)), ((), ()))
    outs = []
    for h in range(NH):
        qh = q[:, h * HD:(h + 1) * HD]
        kh = k_sel[:, h * HD:(h + 1) * HD]
        vh = v_sel[:, h * HD:(h + 1) * HD]
        lg = lax.dot_general(qh, kh, dn, preferred_element_type=jnp.float32)
        m = jnp.max(lg, axis=1, keepdims=True)
        e = jnp.exp(lg - m)
        p = e / jnp.sum(e, axis=1, keepdims=True)
        outs.append(jnp.dot(p, vh, preferred_element_type=jnp.float32))
    return jnp.concatenate(outs, axis=1).reshape(WS, WS, DIM).astype(jnp.bfloat16)


def _attn_body(idx1_ref, idx2_ref, q1_ref, q2_ref, kvp1_ref, kvp2_ref,
               a1_ref, a2_ref):
    i = pl.program_id(0)
    for q_ref, kvp_ref, idx_ref, a_ref in (
            (q2_ref, kvp1_ref, idx1_ref, a1_ref),
            (q1_ref, kvp2_ref, idx2_ref, a2_ref)):
        wins = [_attend_one(q_ref, kvp_ref, idx_ref, i * NWIN + c, c)
                for c in range(NWIN)]
        a_ref[0] = jnp.concatenate(wins, axis=1)


def _attention(q1, q2, kvp1, kvp2, idx1, idx2):
    f32 = jnp.float32
    band = lambda i: (i, 0, 0)
    aband = lambda i: (0, i, 0, 0)
    full3 = lambda i: (0, 0, 0)
    outs = (jax.ShapeDtypeStruct((1, H, H, DIM), jnp.bfloat16),
            jax.ShapeDtypeStruct((1, H, H, DIM), jnp.bfloat16))
    in_specs = [
        pl.BlockSpec(memory_space=pltpu.SMEM),
        pl.BlockSpec(memory_space=pltpu.SMEM),
        pl.BlockSpec((NWIN, W2, QK), band),
        pl.BlockSpec((NWIN, W2, QK), band),
        pl.BlockSpec((P2, NKV, 2 * QK), full3),
        pl.BlockSpec((P2, NKV, 2 * QK), full3),
    ]
    out_specs = (pl.BlockSpec((1, WS, H, DIM), aband),
                 pl.BlockSpec((1, WS, H, DIM), aband))
    return pl.pallas_call(
        _attn_body,
        grid=(NWIN,),
        in_specs=in_specs,
        out_specs=out_specs,
        out_shape=outs,
        interpret=_INTERPRET,
    )(idx1, idx2, q1, q2, kvp1, kvp2)


def _lepe_band(i, x_ref, xu_ref, xd_ref, wv, w9, lepe_b):
    xb = jnp.transpose(x_ref[0], (0, 2, 1))
    vb = jnp.dot(xb.reshape(BH * H, DIM), wv,
                 preferred_element_type=jnp.float32).reshape(BH, H, DIM)
    xu = jnp.transpose(xu_ref[0, 0], (1, 0))
    vu = jnp.dot(xu, wv, preferred_element_type=jnp.float32)
    vu = jnp.where(i == 0, 0.0, vu)[None]
    xd = jnp.transpose(xd_ref[0, 0], (1, 0))
    vd = jnp.dot(xd, wv, preferred_element_type=jnp.float32)
    vd = jnp.where(i == BANDS - 1, 0.0, vd)[None]
    v_ext = jnp.concatenate([vu, vb, vd], axis=0)
    zc = jnp.zeros((BH + 2, 1, DIM), jnp.float32)
    v_pad = jnp.concatenate([zc, v_ext, zc], axis=1)
    acc = jnp.broadcast_to(lepe_b.reshape(1, 1, DIM), (BH, H, DIM))
    for ky in range(3):
        for kx in range(3):
            acc = acc + w9[ky, kx] * v_pad[ky:ky + BH, kx:kx + H, :]
    return acc


def _final_body(a_ref, x_ref, xu_ref, xd_ref,
                wv_ref, w9_ref, lb_ref, wo_ref, wob_ref, o_ref):
    i = pl.program_id(0)
    lepe = _lepe_band(i, x_ref, xu_ref, xd_ref, wv_ref[...], w9_ref[...],
                      lb_ref[...])
    s = a_ref[0].astype(jnp.float32).reshape(BH * H, DIM) + lepe.reshape(BH * H, DIM)
    o = jnp.dot(s, wo_ref[...], preferred_element_type=jnp.float32) + wob_ref[...]
    o_ref[0] = jnp.transpose(o.reshape(BH, H, DIM), (0, 2, 1))


def _final(a_img, x, wv, w9, lepe_b, wo_w, wo_b):
    f32 = jnp.float32
    up = lambda i: (0, jnp.maximum(i * BH - 1, 0), 0, 0)
    dn = lambda i: (0, jnp.minimum((i + 1) * BH, H - 1), 0, 0)
    outs = jax.ShapeDtypeStruct((BANDS, BH, DIM, H), f32)
    bspec = pl.BlockSpec((1, BH, H, DIM), lambda i: (i, 0, 0, 0))
    ospec = pl.BlockSpec((1, BH, DIM, H), lambda i: (i, 0, 0, 0))
    xspec = pl.BlockSpec((1, BH, DIM, H), lambda i: (0, i, 0, 0))
    rspec_u = pl.BlockSpec((1, 1, DIM, H), up)
    rspec_d = pl.BlockSpec((1, 1, DIM, H), dn)
    full2 = lambda i: (0, 0)
    in_specs = [
        bspec,
        xspec, rspec_u, rspec_d,
        pl.BlockSpec((DIM, DIM), full2),
        pl.BlockSpec((3, 3, DIM), lambda i: (0, 0, 0)),
        pl.BlockSpec((1, DIM), full2),
        pl.BlockSpec((DIM, DIM), full2),
        pl.BlockSpec((1, DIM), full2),
    ]
    return pl.pallas_call(
        _final_body,
        grid=(BANDS,),
        in_specs=in_specs,
        out_specs=ospec,
        out_shape=outs,
        interpret=_INTERPRET,
    )(a_img.reshape(BANDS, BH, H, DIM), x, x, x, wv, w9,
      lepe_b.reshape(1, -1), wo_w, wo_b.reshape(1, -1))


def kernel(x1, x2, qkv_w, qkv_b, wo_w, wo_b, lepe_w, lepe_b):
    pmat = _pool_matrix()
    x1t = jnp.transpose(x1, (0, 1, 3, 2))
    x2t = jnp.transpose(x2, (0, 1, 3, 2))
    q1, q2, kvp1, kvp2, qm1, km1, qm2, km2 = _stage_a(
        x1t, x2t, qkv_w, qkv_b, pmat)
    idx1, idx2 = _router(qm1.reshape(P2, QK), km1.reshape(P2, QK),
                         qm2.reshape(P2, QK), km2.reshape(P2, QK))
    a1, a2 = _attention(q1, q2, kvp1, kvp2, idx1, idx2)
    wv = qkv_w[:, 2 * QK:]
    w9 = lepe_w[:, 0].transpose(1, 2, 0)
    o1 = _final(a1.reshape(BANDS, BH, H, DIM), x1t, wv, w9, lepe_b, wo_w, wo_b)
    o2 = _final(a2.reshape(BANDS, BH, H, DIM), x2t, wv, w9, lepe_b, wo_w, wo_b)
    o1 = jnp.transpose(o1.reshape(1, H, DIM, H), (0, 1, 3, 2))
    o2 = jnp.transpose(o2.reshape(1, H, DIM, H), (0, 1, 3, 2))
    return o1, o2


# speedup vs baseline: 4.1846x; 1.6557x over previous
import functools

import numpy as np
import jax
import jax.numpy as jnp
from jax import lax
from jax.experimental import pallas as pl
from jax.experimental.pallas import tpu as pltpu

DIM = 192
QK = 192
NWIN = 8
P2 = NWIN * NWIN
WS = 28
W2 = WS * WS
NH = 8
HD = DIM // NH
TOPK = 4
KVW = 4
NKV = KVW * KVW
SCALE = QK ** (-0.5)
H = 224
BH = 14
BANDS = H // BH

_INTERPRET = False


def _pool_matrix():
    p = np.zeros((NKV, W2), np.float32)
    for a in range(KVW):
        for b in range(KVW):
            for r in range(7 * a, 7 * a + 7):
                for c in range(7 * b, 7 * b + 7):
                    p[a * KVW + b, r * WS + c] = 1.0 / 49.0
    return jnp.asarray(p)


def _stage_a_body(x1_ref, x2_ref, w_ref, b_ref, p_ref,
                  q1_ref, q2_ref, kvp1_ref, kvp2_ref,
                  qm1_ref, km1_ref, qm2_ref, km2_ref):
    w = w_ref[...]
    b = b_ref[...]
    pmat = p_ref[...]
    for x_ref, q_ref, kvp_ref, qm_ref, km_ref in (
            (x1_ref, q1_ref, kvp1_ref, qm1_ref, km1_ref),
            (x2_ref, q2_ref, kvp2_ref, qm2_ref, km2_ref)):
        band = jnp.transpose(x_ref[0], (0, 2, 1))
        qs, kvps, qms, kms = [], [], [], []
        for c in range(NWIN):
            xw = band[:, c * WS:(c + 1) * WS, :].reshape(W2, DIM)
            t = jnp.dot(xw, w, preferred_element_type=jnp.float32) + b
            qs.append(t[:, :QK].astype(jnp.bfloat16))
            kvps.append(jnp.dot(pmat, t[:, QK:],
                                preferred_element_type=jnp.float32)
                        .astype(jnp.bfloat16))
            m = jnp.mean(t, axis=0, keepdims=True)
            qms.append(m[:, :QK])
            kms.append(m[:, QK:2 * QK])
        q_ref[...] = jnp.stack(qs)
        kvp_ref[...] = jnp.stack(kvps)
        qm_ref[...] = jnp.stack(qms)
        km_ref[...] = jnp.stack(kms)


def _stage_a(x1, x2, qkv_w, qkv_b, pmat):
    f32 = jnp.float32
    bf16 = jnp.bfloat16
    outs = (
        jax.ShapeDtypeStruct((P2, W2, QK), bf16),
        jax.ShapeDtypeStruct((P2, W2, QK), bf16),
        jax.ShapeDtypeStruct((P2, NKV, 2 * QK), bf16),
        jax.ShapeDtypeStruct((P2, NKV, 2 * QK), bf16),
        jax.ShapeDtypeStruct((P2, 1, QK), f32),
        jax.ShapeDtypeStruct((P2, 1, QK), f32),
        jax.ShapeDtypeStruct((P2, 1, QK), f32),
        jax.ShapeDtypeStruct((P2, 1, QK), f32),
    )
    band = lambda i: (i, 0, 0)
    xband = lambda i: (0, i, 0, 0)
    full2 = lambda i: (0, 0)
    in_specs = [
        pl.BlockSpec((1, WS, DIM, H), xband),
        pl.BlockSpec((1, WS, DIM, H), xband),
        pl.BlockSpec((DIM, 2 * QK + DIM), full2),
        pl.BlockSpec((1, 2 * QK + DIM), full2),
        pl.BlockSpec((NKV, W2), full2),
    ]
    out_specs = (
        pl.BlockSpec((NWIN, W2, QK), band),
        pl.BlockSpec((NWIN, W2, QK), band),
        pl.BlockSpec((NWIN, NKV, 2 * QK), band),
        pl.BlockSpec((NWIN, NKV, 2 * QK), band),
        pl.BlockSpec((NWIN, 1, QK), band),
        pl.BlockSpec((NWIN, 1, QK), band),
        pl.BlockSpec((NWIN, 1, QK), band),
        pl.BlockSpec((NWIN, 1, QK), band),
    )
    return pl.pallas_call(
        _stage_a_body,
        grid=(NWIN,),
        in_specs=in_specs,
        out_specs=out_specs,
        out_shape=outs,
        interpret=_INTERPRET,
    )(x1, x2, qkv_w, qkv_b.reshape(1, -1), pmat)


def _topk_rows(logits):
    colid = lax.broadcasted_iota(jnp.int32, (P2, P2), 1)
    idxs = []
    for _ in range(TOPK):
        mx = jnp.max(logits, axis=1, keepdims=True)
        cand = jnp.where(logits >= mx, colid, P2)
        am = jnp.min(cand, axis=1, keepdims=True)
        idxs.append(am)
        logits = jnp.where(colid == am, -jnp.float32(np.inf), logits)
    return jnp.concatenate(idxs, axis=1)


def _router_body(qm1_ref, km1_ref, qm2_ref, km2_ref, idx1_ref, idx2_ref):
    rowid = lax.broadcasted_iota(jnp.int32, (P2, P2), 0)
    colid = lax.broadcasted_iota(jnp.int32, (P2, P2), 1)
    diag = rowid == colid
    dn = (((1,), (1,)), ((), ()))
    l1 = lax.dot_general(qm2_ref[...] * SCALE, km1_ref[...], dn,
                         preferred_element_type=jnp.float32)
    l1 = jnp.where(diag, 1.0, l1)
    idx1_ref[...] = _topk_rows(l1)
    l2 = lax.dot_general(qm1_ref[...] * SCALE, km2_ref[...], dn,
                         preferred_element_type=jnp.float32)
    l2 = jnp.where(diag, 1.0, l2)
    idx2_ref[...] = _topk_rows(l2)


def _router(qm1, km1, qm2, km2):
    outs = (jax.ShapeDtypeStruct((P2, TOPK), jnp.int32),
            jax.ShapeDtypeStruct((P2, TOPK), jnp.int32))
    spec = pl.BlockSpec((P2, QK), lambda: (0, 0))
    ospec = pl.BlockSpec((P2, TOPK), lambda: (0, 0))
    return pl.pallas_call(
        _router_body,
        grid=(),
        in_specs=[spec] * 4,
        out_specs=(ospec, ospec),
        out_shape=outs,
        interpret=_INTERPRET,
    )(qm1, km1, qm2, km2)


def _head_masks():
    rid = lax.broadcasted_iota(jnp.int32, (DIM, 1), 0) // HD
    cid = lax.broadcasted_iota(jnp.int32, (1, DIM), 1) // HD
    rm = [(rid == h).astype(jnp.float32) for h in range(NH)]
    cm = [(cid == h).astype(jnp.float32) for h in range(NH)]
    return rm, cm


def _attend_one(q_ref, kvp_ref, idx_ref, wi, wslot, rm, cm):
    rows = []
    for j in range(TOPK):
        r = idx_ref[wi, j]
        rows.append(kvp_ref[r])
    kv = jnp.concatenate(rows, axis=0).astype(jnp.float32)
    k_sel = kv[:, :QK]
    v_sel = kv[:, QK:]
    kt = jnp.transpose(k_sel, (1, 0))
    kbd = jnp.concatenate([kt * rm[h] for h in range(NH)], axis=1)
    vbd = jnp.concatenate([v_sel * cm[h] for h in range(NH)], axis=0)
    q = q_ref[wslot].astype(jnp.float32) * SCALE
    lg = jnp.dot(q, kbd, preferred_element_type=jnp.float32)
    m = jnp.max(lg, axis=1, keepdims=True)
    e = jnp.exp(lg - m)
    ps = []
    for h in range(NH):
        eh = e[:, h * 64:(h + 1) * 64]
        ps.append(eh / jnp.sum(eh, axis=1, keepdims=True))
    p = jnp.concatenate(ps, axis=1)
    out = jnp.dot(p, vbd, preferred_element_type=jnp.float32)
    return out.reshape(WS, WS, DIM).astype(jnp.bfloat16)


def _attn_body(idx1_ref, idx2_ref, q1_ref, q2_ref, kvp1_ref, kvp2_ref,
               a1_ref, a2_ref):
    i = pl.program_id(0)
    rm, cm = _head_masks()
    for q_ref, kvp_ref, idx_ref, a_ref in (
            (q2_ref, kvp1_ref, idx1_ref, a1_ref),
            (q1_ref, kvp2_ref, idx2_ref, a2_ref)):
        wins = [_attend_one(q_ref, kvp_ref, idx_ref, i * NWIN + c, c, rm, cm)
                for c in range(NWIN)]
        a_ref[0] = jnp.concatenate(wins, axis=1)


def _attention(q1, q2, kvp1, kvp2, idx1, idx2):
    f32 = jnp.float32
    band = lambda i: (i, 0, 0)
    aband = lambda i: (0, i, 0, 0)
    full3 = lambda i: (0, 0, 0)
    outs = (jax.ShapeDtypeStruct((1, H, H, DIM), jnp.bfloat16),
            jax.ShapeDtypeStruct((1, H, H, DIM), jnp.bfloat16))
    in_specs = [
        pl.BlockSpec(memory_space=pltpu.SMEM),
        pl.BlockSpec(memory_space=pltpu.SMEM),
        pl.BlockSpec((NWIN, W2, QK), band),
        pl.BlockSpec((NWIN, W2, QK), band),
        pl.BlockSpec((P2, NKV, 2 * QK), full3),
        pl.BlockSpec((P2, NKV, 2 * QK), full3),
    ]
    out_specs = (pl.BlockSpec((1, WS, H, DIM), aband),
                 pl.BlockSpec((1, WS, H, DIM), aband))
    return pl.pallas_call(
        _attn_body,
        grid=(NWIN,),
        in_specs=in_specs,
        out_specs=out_specs,
        out_shape=outs,
        interpret=_INTERPRET,
    )(idx1, idx2, q1, q2, kvp1, kvp2)


def _lepe_band(i, x_ref, xu_ref, xd_ref, wv, w9, lepe_b):
    xb = jnp.transpose(x_ref[0], (0, 2, 1))
    vb = jnp.dot(xb.reshape(BH * H, DIM), wv,
                 preferred_element_type=jnp.float32).reshape(BH, H, DIM)
    xu = jnp.transpose(xu_ref[0, 0], (1, 0))
    vu = jnp.dot(xu, wv, preferred_element_type=jnp.float32)
    vu = jnp.where(i == 0, 0.0, vu)[None]
    xd = jnp.transpose(xd_ref[0, 0], (1, 0))
    vd = jnp.dot(xd, wv, preferred_element_type=jnp.float32)
    vd = jnp.where(i == BANDS - 1, 0.0, vd)[None]
    v_ext = jnp.concatenate([vu, vb, vd], axis=0)
    zc = jnp.zeros((BH + 2, 1, DIM), jnp.float32)
    v_pad = jnp.concatenate([zc, v_ext, zc], axis=1)
    acc = jnp.broadcast_to(lepe_b.reshape(1, 1, DIM), (BH, H, DIM))
    for ky in range(3):
        for kx in range(3):
            acc = acc + w9[ky, kx] * v_pad[ky:ky + BH, kx:kx + H, :]
    return acc


def _final_body(a_ref, x_ref, xu_ref, xd_ref,
                wv_ref, w9_ref, lb_ref, wo_ref, wob_ref, o_ref):
    i = pl.program_id(0)
    lepe = _lepe_band(i, x_ref, xu_ref, xd_ref, wv_ref[...], w9_ref[...],
                      lb_ref[...])
    s = a_ref[0].astype(jnp.float32).reshape(BH * H, DIM) + lepe.reshape(BH * H, DIM)
    o = jnp.dot(s, wo_ref[...], preferred_element_type=jnp.float32) + wob_ref[...]
    o_ref[0] = jnp.transpose(o.reshape(BH, H, DIM), (0, 2, 1))


def _final(a_img, x, wv, w9, lepe_b, wo_w, wo_b):
    f32 = jnp.float32
    up = lambda i: (0, jnp.maximum(i * BH - 1, 0), 0, 0)
    dn = lambda i: (0, jnp.minimum((i + 1) * BH, H - 1), 0, 0)
    outs = jax.ShapeDtypeStruct((BANDS, BH, DIM, H), f32)
    bspec = pl.BlockSpec((1, BH, H, DIM), lambda i: (i, 0, 0, 0))
    ospec = pl.BlockSpec((1, BH, DIM, H), lambda i: (i, 0, 0, 0))
    xspec = pl.BlockSpec((1, BH, DIM, H), lambda i: (0, i, 0, 0))
    rspec_u = pl.BlockSpec((1, 1, DIM, H), up)
    rspec_d = pl.BlockSpec((1, 1, DIM, H), dn)
    full2 = lambda i: (0, 0)
    in_specs = [
        bspec,
        xspec, rspec_u, rspec_d,
        pl.BlockSpec((DIM, DIM), full2),
        pl.BlockSpec((3, 3, DIM), lambda i: (0, 0, 0)),
        pl.BlockSpec((1, DIM), full2),
        pl.BlockSpec((DIM, DIM), full2),
        pl.BlockSpec((1, DIM), full2),
    ]
    return pl.pallas_call(
        _final_body,
        grid=(BANDS,),
        in_specs=in_specs,
        out_specs=ospec,
        out_shape=outs,
        interpret=_INTERPRET,
    )(a_img.reshape(BANDS, BH, H, DIM), x, x, x, wv, w9,
      lepe_b.reshape(1, -1), wo_w, wo_b.reshape(1, -1))


def kernel(x1, x2, qkv_w, qkv_b, wo_w, wo_b, lepe_w, lepe_b):
    pmat = _pool_matrix()
    x1t = jnp.transpose(x1, (0, 1, 3, 2))
    x2t = jnp.transpose(x2, (0, 1, 3, 2))
    q1, q2, kvp1, kvp2, qm1, km1, qm2, km2 = _stage_a(
        x1t, x2t, qkv_w, qkv_b, pmat)
    idx1, idx2 = _router(qm1.reshape(P2, QK), km1.reshape(P2, QK),
                         qm2.reshape(P2, QK), km2.reshape(P2, QK))
    a1, a2 = _attention(q1, q2, kvp1, kvp2, idx1, idx2)
    wv = qkv_w[:, 2 * QK:]
    w9 = lepe_w[:, 0].transpose(1, 2, 0)
    o1 = _final(a1.reshape(BANDS, BH, H, DIM), x1t, wv, w9, lepe_b, wo_w, wo_b)
    o2 = _final(a2.reshape(BANDS, BH, H, DIM), x2t, wv, w9, lepe_b, wo_w, wo_b)
    o1 = jnp.transpose(o1.reshape(1, H, DIM, H), (0, 1, 3, 2))
    o2 = jnp.transpose(o2.reshape(1, H, DIM, H), (0, 1, 3, 2))
    return o1, o2
